# Initial kernel scaffold; baseline (speedup 1.0000x reference)
import jax
import jax.numpy as jnp
from jax.experimental import pallas as pl


def kernel(V, E, edges, cells, edge_to_cells, ce_W1, ce_b1, ce_W2, ce_b2, eu_W1, eu_b1, eu_W2, eu_b2, nu_W1, nu_b1, nu_W2, nu_b2):
    raise NotImplementedError("write your pallas kernel here")


# speedup vs baseline: 6.2586x; 6.2586x over previous
import functools

import jax
import jax.numpy as jnp
from jax import lax
from jax.experimental import pallas as pl
from jax.experimental.pallas import tpu as pltpu
from jax.experimental.pallas import tpu_sc as plsc

N = 10000
MC = 20000
ME = 320000
NF = 128
ES = 16

NCORES = 2
NSUB = 16
NW = NCORES * NSUB

CELLS_PAD = 20480
CPW = CELLS_PAD // NW
CSUB = 128
NCSUB = CPW // CSUB

ME_PAD = 327680
EPW = ME_PAD // NW
ESUB = 128
GRP = 1024
NGRP = EPW // GRP
SPW = EPW // ESUB

PAYW = 24
ACC_ROWS = 10240
RPT = ACC_ROWS // NSUB
DUMMY = N


def _mesh():
    return plsc.VectorSubcoreMesh(
        core_axis_name="c", subcore_axis_name="s",
        num_cores=NCORES, num_subcores=NSUB)


_SC_PARAMS = pltpu.CompilerParams(use_tc_tiling_on_sc=False)


def _sc_cell_mean(V2, cellsp):
    @functools.partial(
        pl.kernel,
        out_type=jax.ShapeDtypeStruct((CELLS_PAD, NF), jnp.float32),
        mesh=_mesh(),
        scratch_types=[
            pltpu.VMEM((NCSUB * 3, 128), jnp.int32),
            pltpu.VMEM((3 * CSUB, NF), jnp.float32),
            pltpu.VMEM((CSUB, NF), jnp.float32),
            pltpu.SemaphoreType.DMA,
        ],
        compiler_params=_SC_PARAMS,
    )
    def k(v_hbm, cells_hbm, out_hbm, idx_v, rows_v, out_v, sem):
        wid = lax.axis_index("s") * NCORES + lax.axis_index("c")
        pltpu.sync_copy(cells_hbm.at[wid], idx_v)

        def sub(j, carry):
            d0 = pltpu.async_copy(v_hbm.at[idx_v.at[j * 3 + 0]],
                                  rows_v.at[pl.ds(0, CSUB)], sem)
            d1 = pltpu.async_copy(v_hbm.at[idx_v.at[j * 3 + 1]],
                                  rows_v.at[pl.ds(CSUB, CSUB)], sem)
            d2 = pltpu.async_copy(v_hbm.at[idx_v.at[j * 3 + 2]],
                                  rows_v.at[pl.ds(2 * CSUB, CSUB)], sem)
            d0.wait()
            d1.wait()
            d2.wait()

            def cell(ci, c2):
                for k8 in range(NF // 16):
                    sl = pl.ds(k8 * 16, 16)
                    s = (rows_v[3 * ci, sl] + rows_v[3 * ci + 1, sl]
                         + rows_v[3 * ci + 2, sl])
                    out_v[ci, sl] = s * (1.0 / 3.0)
                return c2

            lax.fori_loop(0, CSUB, cell, 0, unroll=2)
            pltpu.sync_copy(out_v,
                            out_hbm.at[pl.ds(wid * CPW + j * CSUB, CSUB)])
            return carry

        lax.fori_loop(0, NCSUB, sub, 0)

    return k(V2, cellsp)


def _sc_edge(epre, pltab, prtab, idxl, idxr, idx0, idx1):
    out_types = [
        jax.ShapeDtypeStruct((ME_PAD, ES), jnp.float32),
        jax.ShapeDtypeStruct((NCORES, ACC_ROWS, PAYW), jnp.float32),
        jax.ShapeDtypeStruct((NCORES, ACC_ROWS, PAYW), jnp.float32),
    ]

    @functools.partial(
        pl.kernel,
        out_type=out_types,
        mesh=_mesh(),
        scratch_types=[
            pltpu.VMEM((SPW, 128), jnp.int32),
            pltpu.VMEM((SPW, 128), jnp.int32),
            pltpu.VMEM((SPW, 128), jnp.int32),
            pltpu.VMEM((SPW, 128), jnp.int32),
            pltpu.VMEM((GRP, ES), jnp.float32),
            pltpu.VMEM((GRP, ES), jnp.float32),
            pltpu.VMEM((ESUB, ES), jnp.float32),
            pltpu.VMEM((ESUB, ES), jnp.float32),
            pltpu.VMEM((ESUB, PAYW), jnp.float32),
            pltpu.VMEM((RPT, PAYW), jnp.float32),
            pltpu.VMEM_SHARED((ACC_ROWS, PAYW), jnp.float32),
            pltpu.VMEM_SHARED((ACC_ROWS, PAYW), jnp.float32),
            pltpu.SemaphoreType.DMA,
            pltpu.SemaphoreType.DMA,
        ],
        compiler_params=_SC_PARAMS,
    )
    def k(epre_hbm, pltab_hbm, prtab_hbm, il_hbm, ir_hbm, i0_hbm, i1_hbm,
          h_hbm, a0_hbm, a1_hbm,
          il_v, ir_v, i0_v, i1_v, epre_v, hout_v, gl_v, gr_v, pay_v, zb_v,
          acc0, acc1, sem, sem2):
        cid = lax.axis_index("c")
        sid = lax.axis_index("s")
        wid = sid * NCORES + cid

        z16 = jnp.zeros((16,), jnp.float32)

        def zrow(i, c):
            zb_v[i, pl.ds(0, 16)] = z16
            zb_v[i, pl.ds(PAYW - 16, 16)] = z16
            return c

        lax.fori_loop(0, RPT, zrow, 0, unroll=4)
        pltpu.sync_copy(zb_v, acc0.at[pl.ds(sid * RPT, RPT)])
        pltpu.sync_copy(zb_v, acc1.at[pl.ds(sid * RPT, RPT)])

        o16 = jnp.ones((16,), jnp.float32)

        def prow(i, c):
            pay_v[i, pl.ds(PAYW - 16, 16)] = o16
            return c

        lax.fori_loop(0, ESUB, prow, 0, unroll=4)
        plsc.subcore_barrier()

        pltpu.sync_copy(il_hbm.at[wid], il_v)
        pltpu.sync_copy(ir_hbm.at[wid], ir_v)
        pltpu.sync_copy(i0_hbm.at[wid], i0_v)
        pltpu.sync_copy(i1_hbm.at[wid], i1_v)

        def grp(g, carry):
            pltpu.sync_copy(epre_hbm.at[pl.ds(wid * EPW + g * GRP, GRP)],
                            epre_v)
            for j in range(GRP // ESUB):
                sc = g * (GRP // ESUB) + j
                dl = pltpu.async_copy(pltab_hbm.at[il_v.at[sc]], gl_v, sem)
                dr = pltpu.async_copy(prtab_hbm.at[ir_v.at[sc]], gr_v, sem2)
                dl.wait()
                dr.wait()

                def edge(e, c2):
                    x = epre_v[j * ESUB + e, :] + gl_v[e, :] + gr_v[e, :]
                    h = x / (1.0 + jnp.exp(-x))
                    pay_v[e, pl.ds(0, 16)] = h
                    hout_v[j * ESUB + e, :] = h
                    return c2

                lax.fori_loop(0, ESUB, edge, 0, unroll=4)
                pltpu.sync_copy(pay_v, acc0.at[i0_v.at[sc]], add=True)
                pltpu.sync_copy(pay_v, acc1.at[i1_v.at[sc]], add=True)
            pltpu.sync_copy(hout_v,
                            h_hbm.at[pl.ds(wid * EPW + g * GRP, GRP)])
            return carry

        lax.fori_loop(0, NGRP, grp, 0)
        plsc.subcore_barrier()

        pltpu.sync_copy(acc0.at[pl.ds(sid * RPT, RPT)], zb_v)
        pltpu.sync_copy(zb_v, a0_hbm.at[cid, pl.ds(sid * RPT, RPT)])
        pltpu.sync_copy(acc1.at[pl.ds(sid * RPT, RPT)], zb_v)
        pltpu.sync_copy(zb_v, a1_hbm.at[cid, pl.ds(sid * RPT, RPT)])

    return k(epre, pltab, prtab, idxl, idxr, idx0, idx1)


def _silu(x):
    return x / (1.0 + jnp.exp(-x))


def _dot(a, b):
    return jnp.dot(a, b, preferred_element_type=jnp.float32)


def _tc_epre(E2p, w1e, b1):
    TILE = 2048

    def body(e_ref, w_ref, b_ref, o_ref):
        o_ref[...] = _dot(e_ref[...], w_ref[...]) + b_ref[...]

    return pl.pallas_call(
        body,
        grid=(ME_PAD // TILE,),
        in_specs=[
            pl.BlockSpec((TILE, ES), lambda i: (i, 0)),
            pl.BlockSpec((ES, ES), lambda i: (0, 0)),
            pl.BlockSpec((1, ES), lambda i: (0, 0)),
        ],
        out_specs=pl.BlockSpec((TILE, ES), lambda i: (i, 0)),
        out_shape=jax.ShapeDtypeStruct((ME_PAD, ES), jnp.float32),
    )(E2p, w1e, b1)


def _tc_cells(cfin, ce_W1, ce_b1, ce_W2, ce_b2, w1l, w1r):
    TILE = 1024

    def body(x_ref, w1_ref, b1_ref, w2_ref, b2_ref, wl_ref, wr_ref,
             pl_ref, pr_ref):
        h = _silu(_dot(x_ref[...], w1_ref[...]) + b1_ref[...])
        cf = _dot(h, w2_ref[...]) + b2_ref[...]
        pl_ref[...] = _dot(cf, wl_ref[...])
        pr_ref[...] = _dot(cf, wr_ref[...])

    return pl.pallas_call(
        body,
        grid=(CELLS_PAD // TILE,),
        in_specs=[
            pl.BlockSpec((TILE, NF), lambda i: (i, 0)),
            pl.BlockSpec((NF, NF), lambda i: (0, 0)),
            pl.BlockSpec((1, NF), lambda i: (0, 0)),
            pl.BlockSpec((NF, NF), lambda i: (0, 0)),
            pl.BlockSpec((1, NF), lambda i: (0, 0)),
            pl.BlockSpec((NF, ES), lambda i: (0, 0)),
            pl.BlockSpec((NF, ES), lambda i: (0, 0)),
        ],
        out_specs=[
            pl.BlockSpec((TILE, ES), lambda i: (i, 0)),
            pl.BlockSpec((TILE, ES), lambda i: (i, 0)),
        ],
        out_shape=[
            jax.ShapeDtypeStruct((CELLS_PAD, ES), jnp.float32),
            jax.ShapeDtypeStruct((CELLS_PAD, ES), jnp.float32),
        ],
    )(cfin, ce_W1, ce_b1, ce_W2, ce_b2, w1l, w1r)


def _tc_edge_out(h, eu_W2, eu_b2):
    TILE = 2000

    def body(h_ref, w_ref, b_ref, o_ref):
        o_ref[...] = _dot(h_ref[...], w_ref[...]) + b_ref[...]

    return pl.pallas_call(
        body,
        grid=(ME // TILE,),
        in_specs=[
            pl.BlockSpec((TILE, ES), lambda i: (i, 0)),
            pl.BlockSpec((ES, ES), lambda i: (0, 0)),
            pl.BlockSpec((1, ES), lambda i: (0, 0)),
        ],
        out_specs=pl.BlockSpec((TILE, ES), lambda i: (i, 0)),
        out_shape=jax.ShapeDtypeStruct((ME, ES), jnp.float32),
    )(h, eu_W2, eu_b2)


def _tc_nodes(V2, a0d, a1d, p0, q0, p1, q1, w1v, b1, w2, b2):
    TILE = 1000

    def body(v_ref, a0_ref, a1_ref, p0_ref, q0_ref, p1_ref, q1_ref,
             w1_ref, b1_ref, w2_ref, b2_ref, o_ref):
        s0 = a0_ref[0] + a0_ref[1]
        s1 = a1_ref[0] + a1_ref[1]
        c0 = s0[:, 16:17]
        c1 = s1[:, 16:17]
        hm0 = s0[:, 0:16] / jnp.maximum(c0, 1.0)
        hm1 = s1[:, 0:16] / jnp.maximum(c1, 1.0)
        t0 = jnp.where(c0 > 0, _dot(hm0, p0_ref[...]) + q0_ref[...], 0.0)
        t1 = jnp.where(c1 > 0, _dot(hm1, p1_ref[...]) + q1_ref[...], 0.0)
        pre = _dot(v_ref[...], w1_ref[...]) + t0 + t1 + b1_ref[...]
        o_ref[...] = _dot(_silu(pre), w2_ref[...]) + b2_ref[...]

    return pl.pallas_call(
        body,
        grid=(N // TILE,),
        in_specs=[
            pl.BlockSpec((TILE, NF), lambda i: (i, 0)),
            pl.BlockSpec((NCORES, TILE, PAYW), lambda i: (0, i, 0)),
            pl.BlockSpec((NCORES, TILE, PAYW), lambda i: (0, i, 0)),
            pl.BlockSpec((ES, NF), lambda i: (0, 0)),
            pl.BlockSpec((1, NF), lambda i: (0, 0)),
            pl.BlockSpec((ES, NF), lambda i: (0, 0)),
            pl.BlockSpec((1, NF), lambda i: (0, 0)),
            pl.BlockSpec((NF, NF), lambda i: (0, 0)),
            pl.BlockSpec((1, NF), lambda i: (0, 0)),
            pl.BlockSpec((NF, NF), lambda i: (0, 0)),
            pl.BlockSpec((1, NF), lambda i: (0, 0)),
        ],
        out_specs=pl.BlockSpec((TILE, NF), lambda i: (i, 0)),
        out_shape=jax.ShapeDtypeStruct((N, NF), jnp.float32),
    )(V2, a0d, a1d, p0, q0, p1, q1, w1v, b1, w2, b2)


def kernel(V, E, edges, cells, edge_to_cells,
           ce_W1, ce_b1, ce_W2, ce_b2,
           eu_W1, eu_b1, eu_W2, eu_b2,
           nu_W1, nu_b1, nu_W2, nu_b2):
    i32 = jnp.int32
    V2 = V.reshape(N, NF)
    E2 = E.reshape(ME, ES)

    cells_flat = cells.reshape(-1).astype(i32)
    cellsp = jnp.pad(cells_flat, (0, CELLS_PAD * 3 - MC * 3)
                     ).reshape(NW, NCSUB * 3, 128)

    lidx = edge_to_cells[0, :, 0].astype(i32)
    ridx = edge_to_cells[0, :, 1].astype(i32)
    lidx2 = jnp.where(lidx >= 0, lidx, ridx)
    ridx2 = jnp.where(ridx >= 0, ridx, lidx)
    pad_e = ME_PAD - ME
    idxl = jnp.pad(lidx2, (0, pad_e)).reshape(NW, SPW, 128)
    idxr = jnp.pad(ridx2, (0, pad_e)).reshape(NW, SPW, 128)
    idx0 = jnp.pad(edges[0, :, 0].astype(i32), (0, pad_e),
                   constant_values=DUMMY).reshape(NW, SPW, 128)
    idx1 = jnp.pad(edges[0, :, 1].astype(i32), (0, pad_e),
                   constant_values=DUMMY).reshape(NW, SPW, 128)
    E2p = jnp.pad(E2, ((0, pad_e), (0, 0)))

    w1e = eu_W1[0:ES]
    w1l = eu_W1[ES:ES + NF]
    w1r = eu_W1[ES + NF:]
    eu_b1r = eu_b1.reshape(1, ES)
    we0 = nu_W1[NF:NF + ES // 2]
    we1 = nu_W1[NF + ES // 2:]
    p0 = eu_W2[:, 0:ES // 2] @ we0
    q0 = (eu_b2[0:ES // 2] @ we0).reshape(1, NF)
    p1 = eu_W2[:, ES // 2:] @ we1
    q1 = (eu_b2[ES // 2:] @ we1).reshape(1, NF)
    w1v = nu_W1[0:NF]

    epre = _tc_epre(E2p, w1e, eu_b1r)
    cfin = _sc_cell_mean(V2, cellsp)

    pltab, prtab = _tc_cells(cfin, ce_W1, ce_b1.reshape(1, NF),
                             ce_W2, ce_b2.reshape(1, NF), w1l, w1r)

    h, a0d, a1d = _sc_edge(epre, pltab, prtab, idxl, idxr, idx0, idx1)

    edge_emb = _tc_edge_out(h[:ME], eu_W2, eu_b2.reshape(1, ES))

    node_emb = _tc_nodes(V2, a0d, a1d, p0, q0, p1, q1, w1v,
                         nu_b1.reshape(1, NF), nu_W2, nu_b2.reshape(1, NF))

    return (node_emb.reshape(1, N, NF), edge_emb.reshape(1, ME, ES))


# speedup vs baseline: 10.1258x; 1.6179x over previous
import functools

import jax
import jax.numpy as jnp
from jax import lax
from jax.experimental import pallas as pl
from jax.experimental.pallas import tpu as pltpu
from jax.experimental.pallas import tpu_sc as plsc

N = 10000
MC = 20000
ME = 320000
NF = 128
ES = 16
C_CORNERS = 3

NCORES = 2
NSUB = 16
NW = NCORES * NSUB

CELLS_PAD = 20480
CPW = CELLS_PAD // NW
CSUB = 128
NCSUB = CPW // CSUB

ME_PAD = 327680
EPW = ME_PAD // NW
ESUB = 128
GRP = 1024
NGRP = EPW // GRP
SPW = EPW // ESUB

PAYW = 24
ACC_ROWS = 10240
RPT = ACC_ROWS // NSUB
DUMMY = N


def _mesh():
    return plsc.VectorSubcoreMesh(
        core_axis_name="c", subcore_axis_name="s",
        num_cores=NCORES, num_subcores=NSUB)


_SC_PARAMS = pltpu.CompilerParams(use_tc_tiling_on_sc=False)


def _sc_cell_mean(V2, cellsc):
    @functools.partial(
        pl.kernel,
        out_type=jax.ShapeDtypeStruct((CELLS_PAD, NF), jnp.float32),
        mesh=_mesh(),
        scratch_types=[
            pltpu.VMEM((3, 128), jnp.int32),
            pltpu.VMEM((CSUB, NF), jnp.float32),
            pltpu.VMEM((CSUB, NF), jnp.float32),
            pltpu.VMEM((CSUB, NF), jnp.float32),
            pltpu.VMEM_SHARED((N, NF), jnp.float32),
            pltpu.SemaphoreType.DMA,
        ],
        compiler_params=_SC_PARAMS,
    )
    def k(v_hbm, cells_hbm, out_hbm, idx_v, r0_v, r1_v, r2_v, v_sp, sem):
        cid = lax.axis_index("c")
        sid = lax.axis_index("s")
        wid = sid * NCORES + cid
        pltpu.sync_copy(v_hbm.at[pl.ds(sid * (N // NSUB), N // NSUB)],
                        v_sp.at[pl.ds(sid * (N // NSUB), N // NSUB)])
        plsc.subcore_barrier()

        def sub(j, carry):
            pltpu.sync_copy(cells_hbm.at[wid, j], idx_v)
            d0 = pltpu.async_copy(v_sp.at[idx_v.at[0]], r0_v, sem)
            d1 = pltpu.async_copy(v_sp.at[idx_v.at[1]], r1_v, sem)
            d2 = pltpu.async_copy(v_sp.at[idx_v.at[2]], r2_v, sem)
            d0.wait()
            d1.wait()
            d2.wait()

            def cell(ci, c2):
                for k8 in range(NF // 16):
                    sl = pl.ds(k8 * 16, 16)
                    r0_v[ci, sl] = (r0_v[ci, sl] + r1_v[ci, sl]
                                    + r2_v[ci, sl]) * (1.0 / 3.0)
                return c2

            lax.fori_loop(0, CSUB, cell, 0, unroll=2)
            pltpu.sync_copy(r0_v,
                            out_hbm.at[pl.ds(wid * CPW + j * CSUB, CSUB)])
            return carry

        lax.fori_loop(0, NCSUB, sub, 0)

    return k(V2, cellsc)


def _sc_edge(epre_pk, pltab, prtab, idxl, idxr, idx0, idx1):
    RPG = GRP // 8
    RPW = EPW // 8
    out_types = [
        jax.ShapeDtypeStruct((ME_PAD // 8, 128), jnp.float32),
        jax.ShapeDtypeStruct((NCORES, ACC_ROWS, PAYW), jnp.float32),
        jax.ShapeDtypeStruct((NCORES, ACC_ROWS, PAYW), jnp.float32),
    ]

    @functools.partial(
        pl.kernel,
        out_type=out_types,
        mesh=_mesh(),
        scratch_types=[
            pltpu.VMEM((GRP // ESUB, 128), jnp.int32),
            pltpu.VMEM((GRP // ESUB, 128), jnp.int32),
            pltpu.VMEM((GRP // ESUB, 128), jnp.int32),
            pltpu.VMEM((GRP // ESUB, 128), jnp.int32),
            pltpu.VMEM((RPG, 128), jnp.float32),
            pltpu.VMEM((RPG, 128), jnp.float32),
            pltpu.VMEM((ESUB, ES), jnp.float32),
            pltpu.VMEM((ESUB, ES), jnp.float32),
            pltpu.VMEM((ESUB, PAYW), jnp.float32),
            pltpu.VMEM((RPT // 5, PAYW), jnp.float32),
            pltpu.VMEM_SHARED((ACC_ROWS, PAYW), jnp.float32),
            pltpu.VMEM_SHARED((ACC_ROWS, PAYW), jnp.float32),
            pltpu.VMEM_SHARED((MC, ES), jnp.float32),
            pltpu.VMEM_SHARED((MC, ES), jnp.float32),
            pltpu.SemaphoreType.DMA,
            pltpu.SemaphoreType.DMA,
        ],
        compiler_params=_SC_PARAMS,
    )
    def k(epre_hbm, pltab_hbm, prtab_hbm, il_hbm, ir_hbm, i0_hbm, i1_hbm,
          h_hbm, a0_hbm, a1_hbm,
          il_v, ir_v, i0_v, i1_v, epre_v, hout_v, gl_v, gr_v, pay_v, zb_v,
          acc0, acc1, pl_sp, pr_sp, sem, sem2):
        cid = lax.axis_index("c")
        sid = lax.axis_index("s")
        wid = sid * NCORES + cid
        ZCH = RPT // 5

        pltpu.sync_copy(pltab_hbm.at[pl.ds(sid * (MC // NSUB), MC // NSUB)],
                        pl_sp.at[pl.ds(sid * (MC // NSUB), MC // NSUB)])
        pltpu.sync_copy(prtab_hbm.at[pl.ds(sid * (MC // NSUB), MC // NSUB)],
                        pr_sp.at[pl.ds(sid * (MC // NSUB), MC // NSUB)])

        z16 = jnp.zeros((16,), jnp.float32)

        def zrow(i, c):
            zb_v[i, pl.ds(0, 16)] = z16
            zb_v[i, pl.ds(PAYW - 16, 16)] = z16
            return c

        lax.fori_loop(0, ZCH, zrow, 0, unroll=4)

        def zch(i, c):
            pltpu.sync_copy(zb_v, acc0.at[pl.ds(sid * RPT + i * ZCH, ZCH)])
            pltpu.sync_copy(zb_v, acc1.at[pl.ds(sid * RPT + i * ZCH, ZCH)])
            return c

        lax.fori_loop(0, 5, zch, 0)

        o16 = jnp.ones((16,), jnp.float32)

        def prow(i, c):
            pay_v[i, pl.ds(PAYW - 16, 16)] = o16
            return c

        lax.fori_loop(0, ESUB, prow, 0, unroll=4)
        plsc.subcore_barrier()

        def grp(g, carry):
            pltpu.sync_copy(epre_hbm.at[pl.ds(wid * RPW + g * RPG, RPG)],
                            epre_v)
            pltpu.sync_copy(il_hbm.at[wid, pl.ds(g * 8, 8)], il_v)
            pltpu.sync_copy(ir_hbm.at[wid, pl.ds(g * 8, 8)], ir_v)
            pltpu.sync_copy(i0_hbm.at[wid, pl.ds(g * 8, 8)], i0_v)
            pltpu.sync_copy(i1_hbm.at[wid, pl.ds(g * 8, 8)], i1_v)
            for j in range(GRP // ESUB):
                dl = pltpu.async_copy(pl_sp.at[il_v.at[j]], gl_v, sem)
                dr = pltpu.async_copy(pr_sp.at[ir_v.at[j]], gr_v, sem2)
                dl.wait()
                dr.wait()

                def edge(r2, c2):
                    for kk in range(8):
                        es = r2 * 8 + kk
                        sl = pl.ds(kk * 16, 16)
                        x = epre_v[j * 16 + r2, sl] + gl_v[es, :] + gr_v[es, :]
                        h = x / (1.0 + jnp.exp(-x))
                        pay_v[es, pl.ds(0, 16)] = h
                        hout_v[j * 16 + r2, sl] = h
                    return c2

                lax.fori_loop(0, ESUB // 8, edge, 0, unroll=2)
                pltpu.sync_copy(pay_v, acc0.at[i0_v.at[j]], add=True)
                pltpu.sync_copy(pay_v, acc1.at[i1_v.at[j]], add=True)
            pltpu.sync_copy(hout_v,
                            h_hbm.at[pl.ds(wid * RPW + g * RPG, RPG)])
            return carry

        lax.fori_loop(0, NGRP, grp, 0)
        plsc.subcore_barrier()

        def dch(i, c):
            pltpu.sync_copy(acc0.at[pl.ds(sid * RPT + i * ZCH, ZCH)], zb_v)
            pltpu.sync_copy(zb_v, a0_hbm.at[cid,
                                            pl.ds(sid * RPT + i * ZCH, ZCH)])
            pltpu.sync_copy(acc1.at[pl.ds(sid * RPT + i * ZCH, ZCH)], zb_v)
            pltpu.sync_copy(zb_v, a1_hbm.at[cid,
                                            pl.ds(sid * RPT + i * ZCH, ZCH)])
            return c

        lax.fori_loop(0, 5, dch, 0)

    return k(epre_pk, pltab, prtab, idxl, idxr, idx0, idx1)


def _silu(x):
    return x / (1.0 + jnp.exp(-x))


def _dot(a, b):
    return jnp.dot(a, b, preferred_element_type=jnp.float32)


def _tc_epre(Epk, w1e_blk, b1_blk):
    TILE = 4096
    ROWS = ME_PAD // 8

    def body(e_ref, w_ref, b_ref, o_ref):
        o_ref[...] = _dot(e_ref[...], w_ref[...]) + b_ref[...]

    return pl.pallas_call(
        body,
        grid=(ROWS // TILE,),
        in_specs=[
            pl.BlockSpec((TILE, 128), lambda i: (i, 0)),
            pl.BlockSpec((128, 128), lambda i: (0, 0)),
            pl.BlockSpec((1, 128), lambda i: (0, 0)),
        ],
        out_specs=pl.BlockSpec((TILE, 128), lambda i: (i, 0)),
        out_shape=jax.ShapeDtypeStruct((ROWS, 128), jnp.float32),
    )(Epk, w1e_blk, b1_blk)


def _tc_cells(cfin, ce_W1, ce_b1, ce_W2, ce_b2, w1l, w1r):
    TILE = 1024

    def body(x_ref, w1_ref, b1_ref, w2_ref, b2_ref, wl_ref, wr_ref,
             pl_ref, pr_ref):
        h = _silu(_dot(x_ref[...], w1_ref[...]) + b1_ref[...])
        cf = _dot(h, w2_ref[...]) + b2_ref[...]
        pl_ref[...] = _dot(cf, wl_ref[...])
        pr_ref[...] = _dot(cf, wr_ref[...])

    return pl.pallas_call(
        body,
        grid=(CELLS_PAD // TILE,),
        in_specs=[
            pl.BlockSpec((TILE, NF), lambda i: (i, 0)),
            pl.BlockSpec((NF, NF), lambda i: (0, 0)),
            pl.BlockSpec((1, NF), lambda i: (0, 0)),
            pl.BlockSpec((NF, NF), lambda i: (0, 0)),
            pl.BlockSpec((1, NF), lambda i: (0, 0)),
            pl.BlockSpec((NF, ES), lambda i: (0, 0)),
            pl.BlockSpec((NF, ES), lambda i: (0, 0)),
        ],
        out_specs=[
            pl.BlockSpec((TILE, ES), lambda i: (i, 0)),
            pl.BlockSpec((TILE, ES), lambda i: (i, 0)),
        ],
        out_shape=[
            jax.ShapeDtypeStruct((CELLS_PAD, ES), jnp.float32),
            jax.ShapeDtypeStruct((CELLS_PAD, ES), jnp.float32),
        ],
    )(cfin, ce_W1, ce_b1, ce_W2, ce_b2, w1l, w1r)


def _tc_edge_out(h_pk, w2_blk, b2_blk):
    TILE = 4096
    ROWS = ME_PAD // 8

    def body(h_ref, w_ref, b_ref, o_ref):
        o_ref[...] = _dot(h_ref[...], w_ref[...]) + b_ref[...]

    return pl.pallas_call(
        body,
        grid=(ROWS // TILE,),
        in_specs=[
            pl.BlockSpec((TILE, 128), lambda i: (i, 0)),
            pl.BlockSpec((128, 128), lambda i: (0, 0)),
            pl.BlockSpec((1, 128), lambda i: (0, 0)),
        ],
        out_specs=pl.BlockSpec((TILE, 128), lambda i: (i, 0)),
        out_shape=jax.ShapeDtypeStruct((ROWS, 128), jnp.float32),
    )(h_pk, w2_blk, b2_blk)


def _tc_nodes(V2, a0d, a1d, p0, q0, p1, q1, w1v, b1, w2, b2):
    TILE = 1000

    def body(v_ref, a0_ref, a1_ref, p0_ref, q0_ref, p1_ref, q1_ref,
             w1_ref, b1_ref, w2_ref, b2_ref, o_ref):
        s0 = a0_ref[0] + a0_ref[1]
        s1 = a1_ref[0] + a1_ref[1]
        c0 = s0[:, 16:17]
        c1 = s1[:, 16:17]
        hm0 = s0[:, 0:16] / jnp.maximum(c0, 1.0)
        hm1 = s1[:, 0:16] / jnp.maximum(c1, 1.0)
        t0 = jnp.where(c0 > 0, _dot(hm0, p0_ref[...]) + q0_ref[...], 0.0)
        t1 = jnp.where(c1 > 0, _dot(hm1, p1_ref[...]) + q1_ref[...], 0.0)
        pre = _dot(v_ref[...], w1_ref[...]) + t0 + t1 + b1_ref[...]
        o_ref[...] = _dot(_silu(pre), w2_ref[...]) + b2_ref[...]

    return pl.pallas_call(
        body,
        grid=(N // TILE,),
        in_specs=[
            pl.BlockSpec((TILE, NF), lambda i: (i, 0)),
            pl.BlockSpec((NCORES, TILE, PAYW), lambda i: (0, i, 0)),
            pl.BlockSpec((NCORES, TILE, PAYW), lambda i: (0, i, 0)),
            pl.BlockSpec((ES, NF), lambda i: (0, 0)),
            pl.BlockSpec((1, NF), lambda i: (0, 0)),
            pl.BlockSpec((ES, NF), lambda i: (0, 0)),
            pl.BlockSpec((1, NF), lambda i: (0, 0)),
            pl.BlockSpec((NF, NF), lambda i: (0, 0)),
            pl.BlockSpec((1, NF), lambda i: (0, 0)),
            pl.BlockSpec((NF, NF), lambda i: (0, 0)),
            pl.BlockSpec((1, NF), lambda i: (0, 0)),
        ],
        out_specs=pl.BlockSpec((TILE, NF), lambda i: (i, 0)),
        out_shape=jax.ShapeDtypeStruct((N, NF), jnp.float32),
    )(V2, a0d, a1d, p0, q0, p1, q1, w1v, b1, w2, b2)


def kernel(V, E, edges, cells, edge_to_cells,
           ce_W1, ce_b1, ce_W2, ce_b2,
           eu_W1, eu_b1, eu_W2, eu_b2,
           nu_W1, nu_b1, nu_W2, nu_b2):
    i32 = jnp.int32
    V2 = V.reshape(N, NF)
    E2 = E.reshape(ME, ES)

    cells2 = jnp.pad(cells.reshape(MC, C_CORNERS).astype(i32),
                     ((0, CELLS_PAD - MC), (0, 0)))
    cellsc = cells2.reshape(NW, NCSUB, CSUB, C_CORNERS).transpose(0, 1, 3, 2)

    lidx = edge_to_cells[0, :, 0].astype(i32)
    ridx = edge_to_cells[0, :, 1].astype(i32)
    lidx2 = jnp.where(lidx >= 0, lidx, ridx)
    ridx2 = jnp.where(ridx >= 0, ridx, lidx)
    pad_e = ME_PAD - ME
    idxl = jnp.pad(lidx2, (0, pad_e)).reshape(NW, SPW, 128)
    idxr = jnp.pad(ridx2, (0, pad_e)).reshape(NW, SPW, 128)
    idx0 = jnp.pad(edges[0, :, 0].astype(i32), (0, pad_e),
                   constant_values=DUMMY).reshape(NW, SPW, 128)
    idx1 = jnp.pad(edges[0, :, 1].astype(i32), (0, pad_e),
                   constant_values=DUMMY).reshape(NW, SPW, 128)
    Epk = jnp.pad(E2, ((0, pad_e), (0, 0))).reshape(ME_PAD // 8, 128)

    w1e = eu_W1[0:ES]
    w1l = eu_W1[ES:ES + NF]
    w1r = eu_W1[ES + NF:]
    eye8 = jnp.eye(8, dtype=jnp.float32)
    w1e_blk = jnp.kron(eye8, w1e)
    b1_blk = jnp.tile(eu_b1, 8).reshape(1, 128)
    w2_blk = jnp.kron(eye8, eu_W2)
    b2_blk = jnp.tile(eu_b2, 8).reshape(1, 128)
    we0 = nu_W1[NF:NF + ES // 2]
    we1 = nu_W1[NF + ES // 2:]
    p0 = eu_W2[:, 0:ES // 2] @ we0
    q0 = (eu_b2[0:ES // 2] @ we0).reshape(1, NF)
    p1 = eu_W2[:, ES // 2:] @ we1
    q1 = (eu_b2[ES // 2:] @ we1).reshape(1, NF)
    w1v = nu_W1[0:NF]

    epre_pk = _tc_epre(Epk, w1e_blk, b1_blk)
    cfin = _sc_cell_mean(V2, cellsc)

    pltab, prtab = _tc_cells(cfin, ce_W1, ce_b1.reshape(1, NF),
                             ce_W2, ce_b2.reshape(1, NF), w1l, w1r)

    h_pk, a0d, a1d = _sc_edge(epre_pk, pltab, prtab, idxl, idxr, idx0, idx1)

    edge_emb = _tc_edge_out(h_pk, w2_blk, b2_blk).reshape(ME_PAD, ES)[:ME]

    node_emb = _tc_nodes(V2, a0d, a1d, p0, q0, p1, q1, w1v,
                         nu_b1.reshape(1, NF), nu_W2, nu_b2.reshape(1, NF))

    return (node_emb.reshape(1, N, NF), edge_emb.reshape(1, ME, ES))


# speedup vs baseline: 10.6826x; 1.0550x over previous
import functools

import jax
import jax.numpy as jnp
from jax import lax
from jax.experimental import pallas as pl
from jax.experimental.pallas import tpu as pltpu
from jax.experimental.pallas import tpu_sc as plsc

N = 10000
MC = 20000
ME = 320000
NF = 128
ES = 16
C_CORNERS = 3

NCORES = 2
NSUB = 16
NW = NCORES * NSUB

CELLS_PAD = 20480
CPW = CELLS_PAD // NW
CSUB = 128
NCSUB = CPW // CSUB

ME_PAD = 327680
EPW = ME_PAD // NW
ESUB = 128
GRP = 1024
NGRP = EPW // GRP
SPW = EPW // ESUB

PAYW = 24
ACC_ROWS = 10240
RPT = ACC_ROWS // NSUB
DUMMY = N


def _mesh():
    return plsc.VectorSubcoreMesh(
        core_axis_name="c", subcore_axis_name="s",
        num_cores=NCORES, num_subcores=NSUB)


_SC_PARAMS = pltpu.CompilerParams(use_tc_tiling_on_sc=False)


def _sc_cell_mean(V2, cellsc):
    @functools.partial(
        pl.kernel,
        out_type=jax.ShapeDtypeStruct((CELLS_PAD, NF), jnp.float32),
        mesh=_mesh(),
        scratch_types=[
            pltpu.VMEM((3, 128), jnp.int32),
            pltpu.VMEM((CSUB, NF), jnp.float32),
            pltpu.VMEM((CSUB, NF), jnp.float32),
            pltpu.VMEM((CSUB, NF), jnp.float32),
            pltpu.VMEM_SHARED((N, NF), jnp.float32),
            pltpu.SemaphoreType.DMA,
        ],
        compiler_params=_SC_PARAMS,
    )
    def k(v_hbm, cells_hbm, out_hbm, idx_v, r0_v, r1_v, r2_v, v_sp, sem):
        cid = lax.axis_index("c")
        sid = lax.axis_index("s")
        wid = sid * NCORES + cid
        pltpu.sync_copy(v_hbm.at[pl.ds(sid * (N // NSUB), N // NSUB)],
                        v_sp.at[pl.ds(sid * (N // NSUB), N // NSUB)])
        plsc.subcore_barrier()

        def sub(j, carry):
            pltpu.sync_copy(cells_hbm.at[wid, j], idx_v)
            d0 = pltpu.async_copy(v_sp.at[idx_v.at[0]], r0_v, sem)
            d1 = pltpu.async_copy(v_sp.at[idx_v.at[1]], r1_v, sem)
            d2 = pltpu.async_copy(v_sp.at[idx_v.at[2]], r2_v, sem)
            d0.wait()
            d1.wait()
            d2.wait()

            def cell(ci, c2):
                for k8 in range(NF // 16):
                    sl = pl.ds(k8 * 16, 16)
                    r0_v[ci, sl] = (r0_v[ci, sl] + r1_v[ci, sl]
                                    + r2_v[ci, sl]) * (1.0 / 3.0)
                return c2

            lax.fori_loop(0, CSUB, cell, 0, unroll=2)
            pltpu.sync_copy(r0_v,
                            out_hbm.at[pl.ds(wid * CPW + j * CSUB, CSUB)])
            return carry

        lax.fori_loop(0, NCSUB, sub, 0)

    return k(V2, cellsc)


def _sc_edge(epre_pk, pltab, prtab, idxl, idxr, idx0, idx1):
    RPG = GRP // 8
    RPW = EPW // 8
    out_types = [
        jax.ShapeDtypeStruct((ME_PAD // 8, 128), jnp.float32),
        jax.ShapeDtypeStruct((NCORES, ACC_ROWS, PAYW), jnp.float32),
        jax.ShapeDtypeStruct((NCORES, ACC_ROWS, PAYW), jnp.float32),
    ]

    @functools.partial(
        pl.kernel,
        out_type=out_types,
        mesh=_mesh(),
        scratch_types=[
            pltpu.VMEM((GRP // ESUB, 128), jnp.int32),
            pltpu.VMEM((GRP // ESUB, 128), jnp.int32),
            pltpu.VMEM((GRP // ESUB, 128), jnp.int32),
            pltpu.VMEM((GRP // ESUB, 128), jnp.int32),
            pltpu.VMEM((RPG, 128), jnp.float32),
            pltpu.VMEM((RPG, 128), jnp.float32),
            pltpu.VMEM((2, ESUB, ES), jnp.float32),
            pltpu.VMEM((2, ESUB, ES), jnp.float32),
            pltpu.VMEM((2, ESUB, PAYW), jnp.float32),
            pltpu.VMEM((RPT // 5, PAYW), jnp.float32),
            pltpu.VMEM_SHARED((ACC_ROWS, PAYW), jnp.float32),
            pltpu.VMEM_SHARED((ACC_ROWS, PAYW), jnp.float32),
            pltpu.VMEM_SHARED((MC, ES), jnp.float32),
            pltpu.VMEM_SHARED((MC, ES), jnp.float32),
            pltpu.SemaphoreType.DMA,
            pltpu.SemaphoreType.DMA,
            pltpu.SemaphoreType.DMA,
            pltpu.SemaphoreType.DMA,
        ],
        compiler_params=_SC_PARAMS,
    )
    def k(epre_hbm, pltab_hbm, prtab_hbm, il_hbm, ir_hbm, i0_hbm, i1_hbm,
          h_hbm, a0_hbm, a1_hbm,
          il_v, ir_v, i0_v, i1_v, epre_v, hout_v, gl_v, gr_v, pay_v, zb_v,
          acc0, acc1, pl_sp, pr_sp, sem, sem2, sem3, sem4):
        cid = lax.axis_index("c")
        sid = lax.axis_index("s")
        wid = sid * NCORES + cid
        ZCH = RPT // 5

        pltpu.sync_copy(pltab_hbm.at[pl.ds(sid * (MC // NSUB), MC // NSUB)],
                        pl_sp.at[pl.ds(sid * (MC // NSUB), MC // NSUB)])
        pltpu.sync_copy(prtab_hbm.at[pl.ds(sid * (MC // NSUB), MC // NSUB)],
                        pr_sp.at[pl.ds(sid * (MC // NSUB), MC // NSUB)])

        z16 = jnp.zeros((16,), jnp.float32)

        def zrow(i, c):
            zb_v[i, pl.ds(0, 16)] = z16
            zb_v[i, pl.ds(PAYW - 16, 16)] = z16
            return c

        lax.fori_loop(0, ZCH, zrow, 0, unroll=4)

        def zch(i, c):
            pltpu.sync_copy(zb_v, acc0.at[pl.ds(sid * RPT + i * ZCH, ZCH)])
            pltpu.sync_copy(zb_v, acc1.at[pl.ds(sid * RPT + i * ZCH, ZCH)])
            return c

        lax.fori_loop(0, 5, zch, 0)

        o16 = jnp.ones((16,), jnp.float32)

        def prow(i, c):
            pay_v[0, i, pl.ds(PAYW - 16, 16)] = o16
            pay_v[1, i, pl.ds(PAYW - 16, 16)] = o16
            return c

        lax.fori_loop(0, ESUB, prow, 0, unroll=4)
        plsc.subcore_barrier()

        NSC = GRP // ESUB

        def grp(g, carry):
            pltpu.sync_copy(epre_hbm.at[pl.ds(wid * RPW + g * RPG, RPG)],
                            epre_v)
            pltpu.sync_copy(il_hbm.at[wid, pl.ds(g * 8, 8)], il_v)
            pltpu.sync_copy(ir_hbm.at[wid, pl.ds(g * 8, 8)], ir_v)
            pltpu.sync_copy(i0_hbm.at[wid, pl.ds(g * 8, 8)], i0_v)
            pltpu.sync_copy(i1_hbm.at[wid, pl.ds(g * 8, 8)], i1_v)
            gd = [None, None]
            sd = [None, None, None, None]
            gd[0] = (pltpu.async_copy(pl_sp.at[il_v.at[0]], gl_v.at[0], sem),
                     pltpu.async_copy(pr_sp.at[ir_v.at[0]], gr_v.at[0], sem2))
            for j in range(NSC):
                b = j % 2
                dl, dr = gd[b]
                dl.wait()
                dr.wait()
                if j + 1 < NSC:
                    nb = (j + 1) % 2
                    gd[nb] = (
                        pltpu.async_copy(pl_sp.at[il_v.at[j + 1]],
                                         gl_v.at[nb], sem),
                        pltpu.async_copy(pr_sp.at[ir_v.at[j + 1]],
                                         gr_v.at[nb], sem2))
                if j >= 2:
                    sd[2 * (j % 2)].wait()
                    sd[2 * (j % 2) + 1].wait()

                def edge(r2, c2):
                    for kk in range(8):
                        es = r2 * 8 + kk
                        sl = pl.ds(kk * 16, 16)
                        x = (epre_v[j * 16 + r2, sl] + gl_v[b, es, :]
                             + gr_v[b, es, :])
                        h = x / (1.0 + jnp.exp(-x))
                        pay_v[b, es, pl.ds(0, 16)] = h
                        hout_v[j * 16 + r2, sl] = h
                    return c2

                lax.fori_loop(0, ESUB // 8, edge, 0, unroll=2)
                sd[2 * b] = pltpu.async_copy(pay_v.at[b],
                                             acc0.at[i0_v.at[j]], sem3,
                                             add=True)
                sd[2 * b + 1] = pltpu.async_copy(pay_v.at[b],
                                                 acc1.at[i1_v.at[j]], sem4,
                                                 add=True)
            for j in (NSC - 2, NSC - 1):
                sd[2 * (j % 2)].wait()
                sd[2 * (j % 2) + 1].wait()
            pltpu.sync_copy(hout_v,
                            h_hbm.at[pl.ds(wid * RPW + g * RPG, RPG)])
            return carry

        lax.fori_loop(0, NGRP, grp, 0)
        plsc.subcore_barrier()

        def dch(i, c):
            pltpu.sync_copy(acc0.at[pl.ds(sid * RPT + i * ZCH, ZCH)], zb_v)
            pltpu.sync_copy(zb_v, a0_hbm.at[cid,
                                            pl.ds(sid * RPT + i * ZCH, ZCH)])
            pltpu.sync_copy(acc1.at[pl.ds(sid * RPT + i * ZCH, ZCH)], zb_v)
            pltpu.sync_copy(zb_v, a1_hbm.at[cid,
                                            pl.ds(sid * RPT + i * ZCH, ZCH)])
            return c

        lax.fori_loop(0, 5, dch, 0)

    return k(epre_pk, pltab, prtab, idxl, idxr, idx0, idx1)


def _silu(x):
    return x / (1.0 + jnp.exp(-x))


def _dot(a, b):
    return jnp.dot(a, b, preferred_element_type=jnp.float32)


def _tc_epre(Epk, w1e_blk, b1_blk):
    TILE = 4096
    ROWS = ME_PAD // 8

    def body(e_ref, w_ref, b_ref, o_ref):
        o_ref[...] = _dot(e_ref[...], w_ref[...]) + b_ref[...]

    return pl.pallas_call(
        body,
        grid=(ROWS // TILE,),
        in_specs=[
            pl.BlockSpec((TILE, 128), lambda i: (i, 0)),
            pl.BlockSpec((128, 128), lambda i: (0, 0)),
            pl.BlockSpec((1, 128), lambda i: (0, 0)),
        ],
        out_specs=pl.BlockSpec((TILE, 128), lambda i: (i, 0)),
        out_shape=jax.ShapeDtypeStruct((ROWS, 128), jnp.float32),
    )(Epk, w1e_blk, b1_blk)


def _tc_cells(cfin, ce_W1, ce_b1, ce_W2, ce_b2, w1l, w1r):
    TILE = 1024

    def body(x_ref, w1_ref, b1_ref, w2_ref, b2_ref, wl_ref, wr_ref,
             pl_ref, pr_ref):
        h = _silu(_dot(x_ref[...], w1_ref[...]) + b1_ref[...])
        cf = _dot(h, w2_ref[...]) + b2_ref[...]
        pl_ref[...] = _dot(cf, wl_ref[...])
        pr_ref[...] = _dot(cf, wr_ref[...])

    return pl.pallas_call(
        body,
        grid=(CELLS_PAD // TILE,),
        in_specs=[
            pl.BlockSpec((TILE, NF), lambda i: (i, 0)),
            pl.BlockSpec((NF, NF), lambda i: (0, 0)),
            pl.BlockSpec((1, NF), lambda i: (0, 0)),
            pl.BlockSpec((NF, NF), lambda i: (0, 0)),
            pl.BlockSpec((1, NF), lambda i: (0, 0)),
            pl.BlockSpec((NF, ES), lambda i: (0, 0)),
            pl.BlockSpec((NF, ES), lambda i: (0, 0)),
        ],
        out_specs=[
            pl.BlockSpec((TILE, ES), lambda i: (i, 0)),
            pl.BlockSpec((TILE, ES), lambda i: (i, 0)),
        ],
        out_shape=[
            jax.ShapeDtypeStruct((CELLS_PAD, ES), jnp.float32),
            jax.ShapeDtypeStruct((CELLS_PAD, ES), jnp.float32),
        ],
    )(cfin, ce_W1, ce_b1, ce_W2, ce_b2, w1l, w1r)


def _tc_edge_out(h_pk, w2_blk, b2_blk):
    TILE = 4096
    ROWS = ME_PAD // 8

    def body(h_ref, w_ref, b_ref, o_ref):
        o_ref[...] = _dot(h_ref[...], w_ref[...]) + b_ref[...]

    return pl.pallas_call(
        body,
        grid=(ROWS // TILE,),
        in_specs=[
            pl.BlockSpec((TILE, 128), lambda i: (i, 0)),
            pl.BlockSpec((128, 128), lambda i: (0, 0)),
            pl.BlockSpec((1, 128), lambda i: (0, 0)),
        ],
        out_specs=pl.BlockSpec((TILE, 128), lambda i: (i, 0)),
        out_shape=jax.ShapeDtypeStruct((ROWS, 128), jnp.float32),
    )(h_pk, w2_blk, b2_blk)


def _tc_nodes(V2, a0d, a1d, p0, q0, p1, q1, w1v, b1, w2, b2):
    TILE = 1000

    def body(v_ref, a0_ref, a1_ref, p0_ref, q0_ref, p1_ref, q1_ref,
             w1_ref, b1_ref, w2_ref, b2_ref, o_ref):
        s0 = a0_ref[0] + a0_ref[1]
        s1 = a1_ref[0] + a1_ref[1]
        c0 = s0[:, 16:17]
        c1 = s1[:, 16:17]
        hm0 = s0[:, 0:16] / jnp.maximum(c0, 1.0)
        hm1 = s1[:, 0:16] / jnp.maximum(c1, 1.0)
        t0 = jnp.where(c0 > 0, _dot(hm0, p0_ref[...]) + q0_ref[...], 0.0)
        t1 = jnp.where(c1 > 0, _dot(hm1, p1_ref[...]) + q1_ref[...], 0.0)
        pre = _dot(v_ref[...], w1_ref[...]) + t0 + t1 + b1_ref[...]
        o_ref[...] = _dot(_silu(pre), w2_ref[...]) + b2_ref[...]

    return pl.pallas_call(
        body,
        grid=(N // TILE,),
        in_specs=[
            pl.BlockSpec((TILE, NF), lambda i: (i, 0)),
            pl.BlockSpec((NCORES, TILE, PAYW), lambda i: (0, i, 0)),
            pl.BlockSpec((NCORES, TILE, PAYW), lambda i: (0, i, 0)),
            pl.BlockSpec((ES, NF), lambda i: (0, 0)),
            pl.BlockSpec((1, NF), lambda i: (0, 0)),
            pl.BlockSpec((ES, NF), lambda i: (0, 0)),
            pl.BlockSpec((1, NF), lambda i: (0, 0)),
            pl.BlockSpec((NF, NF), lambda i: (0, 0)),
            pl.BlockSpec((1, NF), lambda i: (0, 0)),
            pl.BlockSpec((NF, NF), lambda i: (0, 0)),
            pl.BlockSpec((1, NF), lambda i: (0, 0)),
        ],
        out_specs=pl.BlockSpec((TILE, NF), lambda i: (i, 0)),
        out_shape=jax.ShapeDtypeStruct((N, NF), jnp.float32),
    )(V2, a0d, a1d, p0, q0, p1, q1, w1v, b1, w2, b2)


def kernel(V, E, edges, cells, edge_to_cells,
           ce_W1, ce_b1, ce_W2, ce_b2,
           eu_W1, eu_b1, eu_W2, eu_b2,
           nu_W1, nu_b1, nu_W2, nu_b2):
    i32 = jnp.int32
    V2 = V.reshape(N, NF)
    E2 = E.reshape(ME, ES)

    cells2 = jnp.pad(cells.reshape(MC, C_CORNERS).astype(i32),
                     ((0, CELLS_PAD - MC), (0, 0)))
    cellsc = cells2.reshape(NW, NCSUB, CSUB, C_CORNERS).transpose(0, 1, 3, 2)

    lidx = edge_to_cells[0, :, 0].astype(i32)
    ridx = edge_to_cells[0, :, 1].astype(i32)
    lidx2 = jnp.where(lidx >= 0, lidx, ridx)
    ridx2 = jnp.where(ridx >= 0, ridx, lidx)
    pad_e = ME_PAD - ME
    idxl = jnp.pad(lidx2, (0, pad_e)).reshape(NW, SPW, 128)
    idxr = jnp.pad(ridx2, (0, pad_e)).reshape(NW, SPW, 128)
    idx0 = jnp.pad(edges[0, :, 0].astype(i32), (0, pad_e),
                   constant_values=DUMMY).reshape(NW, SPW, 128)
    idx1 = jnp.pad(edges[0, :, 1].astype(i32), (0, pad_e),
                   constant_values=DUMMY).reshape(NW, SPW, 128)
    Epk = jnp.pad(E2, ((0, pad_e), (0, 0))).reshape(ME_PAD // 8, 128)

    w1e = eu_W1[0:ES]
    w1l = eu_W1[ES:ES + NF]
    w1r = eu_W1[ES + NF:]
    eye8 = jnp.eye(8, dtype=jnp.float32)
    w1e_blk = jnp.kron(eye8, w1e)
    b1_blk = jnp.tile(eu_b1, 8).reshape(1, 128)
    w2_blk = jnp.kron(eye8, eu_W2)
    b2_blk = jnp.tile(eu_b2, 8).reshape(1, 128)
    we0 = nu_W1[NF:NF + ES // 2]
    we1 = nu_W1[NF + ES // 2:]
    p0 = eu_W2[:, 0:ES // 2] @ we0
    q0 = (eu_b2[0:ES // 2] @ we0).reshape(1, NF)
    p1 = eu_W2[:, ES // 2:] @ we1
    q1 = (eu_b2[ES // 2:] @ we1).reshape(1, NF)
    w1v = nu_W1[0:NF]

    epre_pk = _tc_epre(Epk, w1e_blk, b1_blk)
    cfin = _sc_cell_mean(V2, cellsc)

    pltab, prtab = _tc_cells(cfin, ce_W1, ce_b1.reshape(1, NF),
                             ce_W2, ce_b2.reshape(1, NF), w1l, w1r)

    h_pk, a0d, a1d = _sc_edge(epre_pk, pltab, prtab, idxl, idxr, idx0, idx1)

    edge_emb = _tc_edge_out(h_pk, w2_blk, b2_blk).reshape(ME_PAD, ES)[:ME]

    node_emb = _tc_nodes(V2, a0d, a1d, p0, q0, p1, q1, w1v,
                         nu_b1.reshape(1, NF), nu_W2, nu_b2.reshape(1, NF))

    return (node_emb.reshape(1, N, NF), edge_emb.reshape(1, ME, ES))


# speedup vs baseline: 14.0192x; 1.3123x over previous
import functools

import jax
import jax.numpy as jnp
from jax import lax
from jax.experimental import pallas as pl
from jax.experimental.pallas import tpu as pltpu
from jax.experimental.pallas import tpu_sc as plsc

N = 10000
MC = 20000
ME = 320000
NF = 128
ES = 16
C_CORNERS = 3

NCORES = 2
NSUB = 16
NW = NCORES * NSUB

CELLS_PAD = 20480
CPW = CELLS_PAD // NW
CSUB = 128
NCSUB = CPW // CSUB

ME_PAD = 327680
EPW = ME_PAD // NW
ESUB = 128
GRP = 1024
NGRP = EPW // GRP
SPW = EPW // ESUB

PAYW = 24
ACC_ROWS = 10240
RPT = ACC_ROWS // NSUB
DUMMY = N


def _mesh():
    return plsc.VectorSubcoreMesh(
        core_axis_name="c", subcore_axis_name="s",
        num_cores=NCORES, num_subcores=NSUB)


_SC_PARAMS = pltpu.CompilerParams(use_tc_tiling_on_sc=False)


def _sc_cell_mean(V2, cellsc):
    @functools.partial(
        pl.kernel,
        out_type=jax.ShapeDtypeStruct((CELLS_PAD, NF), jnp.float32),
        mesh=_mesh(),
        scratch_types=[
            pltpu.VMEM((3, 128), jnp.int32),
            pltpu.VMEM((CSUB, NF), jnp.float32),
            pltpu.VMEM((CSUB, NF), jnp.float32),
            pltpu.VMEM((CSUB, NF), jnp.float32),
            pltpu.VMEM_SHARED((N, NF), jnp.float32),
            pltpu.SemaphoreType.DMA,
        ],
        compiler_params=_SC_PARAMS,
    )
    def k(v_hbm, cells_hbm, out_hbm, idx_v, r0_v, r1_v, r2_v, v_sp, sem):
        cid = lax.axis_index("c")
        sid = lax.axis_index("s")
        wid = sid * NCORES + cid
        pltpu.sync_copy(v_hbm.at[pl.ds(sid * (N // NSUB), N // NSUB)],
                        v_sp.at[pl.ds(sid * (N // NSUB), N // NSUB)])
        plsc.subcore_barrier()

        def sub(j, carry):
            pltpu.sync_copy(cells_hbm.at[wid, j], idx_v)
            d0 = pltpu.async_copy(v_sp.at[idx_v.at[0]], r0_v, sem)
            d1 = pltpu.async_copy(v_sp.at[idx_v.at[1]], r1_v, sem)
            d2 = pltpu.async_copy(v_sp.at[idx_v.at[2]], r2_v, sem)
            d0.wait()
            d1.wait()
            d2.wait()

            def cell(ci, c2):
                for k8 in range(NF // 16):
                    sl = pl.ds(k8 * 16, 16)
                    r0_v[ci, sl] = (r0_v[ci, sl] + r1_v[ci, sl]
                                    + r2_v[ci, sl]) * (1.0 / 3.0)
                return c2

            lax.fori_loop(0, CSUB, cell, 0, unroll=2)
            pltpu.sync_copy(r0_v,
                            out_hbm.at[pl.ds(wid * CPW + j * CSUB, CSUB)])
            return carry

        lax.fori_loop(0, NCSUB, sub, 0)

    return k(V2, cellsc)


def _sc_edge(epre_pk, pltab, prtab, idxl, idxr):
    RPG = GRP // 8
    RPW = EPW // 8

    @functools.partial(
        pl.kernel,
        out_type=jax.ShapeDtypeStruct((ME_PAD // 8, 128), jnp.float32),
        mesh=_mesh(),
        scratch_types=[
            pltpu.VMEM((GRP // ESUB, 128), jnp.int32),
            pltpu.VMEM((GRP // ESUB, 128), jnp.int32),
            pltpu.VMEM((RPG, 128), jnp.float32),
            pltpu.VMEM((RPG, 128), jnp.float32),
            pltpu.VMEM((2, ESUB, ES), jnp.float32),
            pltpu.VMEM((2, ESUB, ES), jnp.float32),
            pltpu.VMEM_SHARED((MC, ES), jnp.float32),
            pltpu.VMEM_SHARED((MC, ES), jnp.float32),
            pltpu.SemaphoreType.DMA,
            pltpu.SemaphoreType.DMA,
        ],
        compiler_params=_SC_PARAMS,
    )
    def k(epre_hbm, pltab_hbm, prtab_hbm, il_hbm, ir_hbm, x_hbm,
          il_v, ir_v, epre_v, xout_v, gl_v, gr_v, pl_sp, pr_sp, sem, sem2):
        cid = lax.axis_index("c")
        sid = lax.axis_index("s")
        wid = sid * NCORES + cid

        pltpu.sync_copy(pltab_hbm.at[pl.ds(sid * (MC // NSUB), MC // NSUB)],
                        pl_sp.at[pl.ds(sid * (MC // NSUB), MC // NSUB)])
        pltpu.sync_copy(prtab_hbm.at[pl.ds(sid * (MC // NSUB), MC // NSUB)],
                        pr_sp.at[pl.ds(sid * (MC // NSUB), MC // NSUB)])
        plsc.subcore_barrier()

        NSC = GRP // ESUB

        def grp(g, carry):
            pltpu.sync_copy(epre_hbm.at[pl.ds(wid * RPW + g * RPG, RPG)],
                            epre_v)
            pltpu.sync_copy(il_hbm.at[wid, pl.ds(g * 8, 8)], il_v)
            pltpu.sync_copy(ir_hbm.at[wid, pl.ds(g * 8, 8)], ir_v)
            gd = [None, None]
            gd[0] = (pltpu.async_copy(pl_sp.at[il_v.at[0]], gl_v.at[0], sem),
                     pltpu.async_copy(pr_sp.at[ir_v.at[0]], gr_v.at[0], sem2))
            for j in range(NSC):
                b = j % 2
                dl, dr = gd[b]
                dl.wait()
                dr.wait()
                if j + 1 < NSC:
                    nb = (j + 1) % 2
                    gd[nb] = (
                        pltpu.async_copy(pl_sp.at[il_v.at[j + 1]],
                                         gl_v.at[nb], sem),
                        pltpu.async_copy(pr_sp.at[ir_v.at[j + 1]],
                                         gr_v.at[nb], sem2))

                def edge(r2, c2):
                    for kk in range(8):
                        es = r2 * 8 + kk
                        sl = pl.ds(kk * 16, 16)
                        xout_v[j * 16 + r2, sl] = (
                            epre_v[j * 16 + r2, sl] + gl_v[b, es, :]
                            + gr_v[b, es, :])
                    return c2

                lax.fori_loop(0, ESUB // 8, edge, 0, unroll=2)
            pltpu.sync_copy(xout_v,
                            x_hbm.at[pl.ds(wid * RPW + g * RPG, RPG)])
            return carry

        lax.fori_loop(0, NGRP, grp, 0)

    return k(epre_pk, pltab, prtab, idxl, idxr)


def _sc_scatter(h_pk, idx0, idx1):
    RPG = GRP // 8
    RPW = EPW // 8
    out_types = [
        jax.ShapeDtypeStruct((NCORES, ACC_ROWS, PAYW), jnp.float32),
        jax.ShapeDtypeStruct((NCORES, ACC_ROWS, PAYW), jnp.float32),
    ]

    @functools.partial(
        pl.kernel,
        out_type=out_types,
        mesh=_mesh(),
        scratch_types=[
            pltpu.VMEM((GRP // ESUB, 128), jnp.int32),
            pltpu.VMEM((GRP // ESUB, 128), jnp.int32),
            pltpu.VMEM((2, RPG, 128), jnp.float32),
            pltpu.VMEM((2, ESUB, PAYW), jnp.float32),
            pltpu.VMEM((RPT // 5, PAYW), jnp.float32),
            pltpu.VMEM_SHARED((ACC_ROWS, PAYW), jnp.float32),
            pltpu.VMEM_SHARED((ACC_ROWS, PAYW), jnp.float32),
            pltpu.SemaphoreType.DMA,
            pltpu.SemaphoreType.DMA,
            pltpu.SemaphoreType.DMA,
        ],
        compiler_params=_SC_PARAMS,
    )
    def k(h_hbm, i0_hbm, i1_hbm, a0_hbm, a1_hbm,
          i0_v, i1_v, h_v, pay_v, zb_v, acc0, acc1, sem, sem3, sem4):
        cid = lax.axis_index("c")
        sid = lax.axis_index("s")
        wid = sid * NCORES + cid
        ZCH = RPT // 5

        z16 = jnp.zeros((16,), jnp.float32)

        def zrow(i, c):
            zb_v[i, pl.ds(0, 16)] = z16
            zb_v[i, pl.ds(PAYW - 16, 16)] = z16
            return c

        lax.fori_loop(0, ZCH, zrow, 0, unroll=4)

        def zch(i, c):
            pltpu.sync_copy(zb_v, acc0.at[pl.ds(sid * RPT + i * ZCH, ZCH)])
            pltpu.sync_copy(zb_v, acc1.at[pl.ds(sid * RPT + i * ZCH, ZCH)])
            return c

        lax.fori_loop(0, 5, zch, 0)

        o16 = jnp.ones((16,), jnp.float32)

        def prow(i, c):
            pay_v[0, i, pl.ds(PAYW - 16, 16)] = o16
            pay_v[1, i, pl.ds(PAYW - 16, 16)] = o16
            return c

        lax.fori_loop(0, ESUB, prow, 0, unroll=4)
        plsc.subcore_barrier()

        NSC = GRP // ESUB

        def grp(g, carry):
            pltpu.sync_copy(h_hbm.at[pl.ds(wid * RPW + g * RPG, RPG)],
                            h_v.at[0])
            pltpu.sync_copy(i0_hbm.at[wid, pl.ds(g * 8, 8)], i0_v)
            pltpu.sync_copy(i1_hbm.at[wid, pl.ds(g * 8, 8)], i1_v)
            sd = [None, None, None, None]
            for j in range(NSC):
                b = j % 2
                if j >= 2:
                    sd[2 * b].wait()
                    sd[2 * b + 1].wait()

                def edge(r2, c2):
                    for kk in range(8):
                        es = r2 * 8 + kk
                        sl = pl.ds(kk * 16, 16)
                        pay_v[b, es, pl.ds(0, 16)] = h_v[0, j * 16 + r2, sl]
                    return c2

                lax.fori_loop(0, ESUB // 8, edge, 0, unroll=2)
                sd[2 * b] = pltpu.async_copy(pay_v.at[b],
                                             acc0.at[i0_v.at[j]], sem3,
                                             add=True)
                sd[2 * b + 1] = pltpu.async_copy(pay_v.at[b],
                                                 acc1.at[i1_v.at[j]], sem4,
                                                 add=True)
            for j in (NSC - 2, NSC - 1):
                sd[2 * (j % 2)].wait()
                sd[2 * (j % 2) + 1].wait()
            return carry

        lax.fori_loop(0, NGRP, grp, 0)
        plsc.subcore_barrier()

        def dch(i, c):
            pltpu.sync_copy(acc0.at[pl.ds(sid * RPT + i * ZCH, ZCH)], zb_v)
            pltpu.sync_copy(zb_v, a0_hbm.at[cid,
                                            pl.ds(sid * RPT + i * ZCH, ZCH)])
            pltpu.sync_copy(acc1.at[pl.ds(sid * RPT + i * ZCH, ZCH)], zb_v)
            pltpu.sync_copy(zb_v, a1_hbm.at[cid,
                                            pl.ds(sid * RPT + i * ZCH, ZCH)])
            return c

        lax.fori_loop(0, 5, dch, 0)

    return k(h_pk, idx0, idx1)


def _silu(x):
    return x / (1.0 + jnp.exp(-x))


def _dot(a, b):
    return jnp.dot(a, b, preferred_element_type=jnp.float32)


def _tc_epre(Epk, w1e_blk, b1_blk):
    TILE = 4096
    ROWS = ME_PAD // 8

    def body(e_ref, w_ref, b_ref, o_ref):
        o_ref[...] = _dot(e_ref[...], w_ref[...]) + b_ref[...]

    return pl.pallas_call(
        body,
        grid=(ROWS // TILE,),
        in_specs=[
            pl.BlockSpec((TILE, 128), lambda i: (i, 0)),
            pl.BlockSpec((128, 128), lambda i: (0, 0)),
            pl.BlockSpec((1, 128), lambda i: (0, 0)),
        ],
        out_specs=pl.BlockSpec((TILE, 128), lambda i: (i, 0)),
        out_shape=jax.ShapeDtypeStruct((ROWS, 128), jnp.float32),
    )(Epk, w1e_blk, b1_blk)


def _tc_cells(cfin, ce_W1, ce_b1, ce_W2, ce_b2, w1l, w1r):
    TILE = 1024

    def body(x_ref, w1_ref, b1_ref, w2_ref, b2_ref, wl_ref, wr_ref,
             pl_ref, pr_ref):
        h = _silu(_dot(x_ref[...], w1_ref[...]) + b1_ref[...])
        cf = _dot(h, w2_ref[...]) + b2_ref[...]
        pl_ref[...] = _dot(cf, wl_ref[...])
        pr_ref[...] = _dot(cf, wr_ref[...])

    return pl.pallas_call(
        body,
        grid=(CELLS_PAD // TILE,),
        in_specs=[
            pl.BlockSpec((TILE, NF), lambda i: (i, 0)),
            pl.BlockSpec((NF, NF), lambda i: (0, 0)),
            pl.BlockSpec((1, NF), lambda i: (0, 0)),
            pl.BlockSpec((NF, NF), lambda i: (0, 0)),
            pl.BlockSpec((1, NF), lambda i: (0, 0)),
            pl.BlockSpec((NF, ES), lambda i: (0, 0)),
            pl.BlockSpec((NF, ES), lambda i: (0, 0)),
        ],
        out_specs=[
            pl.BlockSpec((TILE, ES), lambda i: (i, 0)),
            pl.BlockSpec((TILE, ES), lambda i: (i, 0)),
        ],
        out_shape=[
            jax.ShapeDtypeStruct((CELLS_PAD, ES), jnp.float32),
            jax.ShapeDtypeStruct((CELLS_PAD, ES), jnp.float32),
        ],
    )(cfin, ce_W1, ce_b1, ce_W2, ce_b2, w1l, w1r)


def _tc_edge_out(x_pk, w2_blk, b2_blk):
    TILE = 4096
    ROWS = ME_PAD // 8

    def body(x_ref, w_ref, b_ref, h_ref, o_ref):
        h = _silu(x_ref[...])
        h_ref[...] = h
        o_ref[...] = _dot(h, w_ref[...]) + b_ref[...]

    return pl.pallas_call(
        body,
        grid=(ROWS // TILE,),
        in_specs=[
            pl.BlockSpec((TILE, 128), lambda i: (i, 0)),
            pl.BlockSpec((128, 128), lambda i: (0, 0)),
            pl.BlockSpec((1, 128), lambda i: (0, 0)),
        ],
        out_specs=[
            pl.BlockSpec((TILE, 128), lambda i: (i, 0)),
            pl.BlockSpec((TILE, 128), lambda i: (i, 0)),
        ],
        out_shape=[
            jax.ShapeDtypeStruct((ROWS, 128), jnp.float32),
            jax.ShapeDtypeStruct((ROWS, 128), jnp.float32),
        ],
    )(x_pk, w2_blk, b2_blk)


def _tc_nodes(V2, a0d, a1d, p0, q0, p1, q1, w1v, b1, w2, b2):
    TILE = 1000

    def body(v_ref, a0_ref, a1_ref, p0_ref, q0_ref, p1_ref, q1_ref,
             w1_ref, b1_ref, w2_ref, b2_ref, o_ref):
        s0 = a0_ref[0] + a0_ref[1]
        s1 = a1_ref[0] + a1_ref[1]
        c0 = s0[:, 16:17]
        c1 = s1[:, 16:17]
        hm0 = s0[:, 0:16] / jnp.maximum(c0, 1.0)
        hm1 = s1[:, 0:16] / jnp.maximum(c1, 1.0)
        t0 = jnp.where(c0 > 0, _dot(hm0, p0_ref[...]) + q0_ref[...], 0.0)
        t1 = jnp.where(c1 > 0, _dot(hm1, p1_ref[...]) + q1_ref[...], 0.0)
        pre = _dot(v_ref[...], w1_ref[...]) + t0 + t1 + b1_ref[...]
        o_ref[...] = _dot(_silu(pre), w2_ref[...]) + b2_ref[...]

    return pl.pallas_call(
        body,
        grid=(N // TILE,),
        in_specs=[
            pl.BlockSpec((TILE, NF), lambda i: (i, 0)),
            pl.BlockSpec((NCORES, TILE, PAYW), lambda i: (0, i, 0)),
            pl.BlockSpec((NCORES, TILE, PAYW), lambda i: (0, i, 0)),
            pl.BlockSpec((ES, NF), lambda i: (0, 0)),
            pl.BlockSpec((1, NF), lambda i: (0, 0)),
            pl.BlockSpec((ES, NF), lambda i: (0, 0)),
            pl.BlockSpec((1, NF), lambda i: (0, 0)),
            pl.BlockSpec((NF, NF), lambda i: (0, 0)),
            pl.BlockSpec((1, NF), lambda i: (0, 0)),
            pl.BlockSpec((NF, NF), lambda i: (0, 0)),
            pl.BlockSpec((1, NF), lambda i: (0, 0)),
        ],
        out_specs=pl.BlockSpec((TILE, NF), lambda i: (i, 0)),
        out_shape=jax.ShapeDtypeStruct((N, NF), jnp.float32),
    )(V2, a0d, a1d, p0, q0, p1, q1, w1v, b1, w2, b2)


def kernel(V, E, edges, cells, edge_to_cells,
           ce_W1, ce_b1, ce_W2, ce_b2,
           eu_W1, eu_b1, eu_W2, eu_b2,
           nu_W1, nu_b1, nu_W2, nu_b2):
    i32 = jnp.int32
    V2 = V.reshape(N, NF)
    E2 = E.reshape(ME, ES)

    cells2 = jnp.pad(cells.reshape(MC, C_CORNERS).astype(i32),
                     ((0, CELLS_PAD - MC), (0, 0)))
    cellsc = cells2.reshape(NW, NCSUB, CSUB, C_CORNERS).transpose(0, 1, 3, 2)

    lidx = edge_to_cells[0, :, 0].astype(i32)
    ridx = edge_to_cells[0, :, 1].astype(i32)
    lidx2 = jnp.where(lidx >= 0, lidx, ridx)
    ridx2 = jnp.where(ridx >= 0, ridx, lidx)
    pad_e = ME_PAD - ME
    idxl = jnp.pad(lidx2, (0, pad_e)).reshape(NW, SPW, 128)
    idxr = jnp.pad(ridx2, (0, pad_e)).reshape(NW, SPW, 128)
    idx0 = jnp.pad(edges[0, :, 0].astype(i32), (0, pad_e),
                   constant_values=DUMMY).reshape(NW, SPW, 128)
    idx1 = jnp.pad(edges[0, :, 1].astype(i32), (0, pad_e),
                   constant_values=DUMMY).reshape(NW, SPW, 128)
    Epk = jnp.pad(E2, ((0, pad_e), (0, 0))).reshape(ME_PAD // 8, 128)

    w1e = eu_W1[0:ES]
    w1l = eu_W1[ES:ES + NF]
    w1r = eu_W1[ES + NF:]
    eye8 = jnp.eye(8, dtype=jnp.float32)
    w1e_blk = jnp.kron(eye8, w1e)
    b1_blk = jnp.tile(eu_b1, 8).reshape(1, 128)
    w2_blk = jnp.kron(eye8, eu_W2)
    b2_blk = jnp.tile(eu_b2, 8).reshape(1, 128)
    we0 = nu_W1[NF:NF + ES // 2]
    we1 = nu_W1[NF + ES // 2:]
    p0 = eu_W2[:, 0:ES // 2] @ we0
    q0 = (eu_b2[0:ES // 2] @ we0).reshape(1, NF)
    p1 = eu_W2[:, ES // 2:] @ we1
    q1 = (eu_b2[ES // 2:] @ we1).reshape(1, NF)
    w1v = nu_W1[0:NF]

    epre_pk = _tc_epre(Epk, w1e_blk, b1_blk)
    cfin = _sc_cell_mean(V2, cellsc)

    pltab, prtab = _tc_cells(cfin, ce_W1, ce_b1.reshape(1, NF),
                             ce_W2, ce_b2.reshape(1, NF), w1l, w1r)

    x_pk = _sc_edge(epre_pk, pltab, prtab, idxl, idxr)

    h_pk, edge_emb_pk = _tc_edge_out(x_pk, w2_blk, b2_blk)
    edge_emb = edge_emb_pk.reshape(ME_PAD, ES)[:ME]

    a0d, a1d = _sc_scatter(h_pk, idx0, idx1)

    node_emb = _tc_nodes(V2, a0d, a1d, p0, q0, p1, q1, w1v,
                         nu_b1.reshape(1, NF), nu_W2, nu_b2.reshape(1, NF))

    return (node_emb.reshape(1, N, NF), edge_emb.reshape(1, ME, ES))


# speedup vs baseline: 15.2609x; 1.0886x over previous
import functools

import jax
import jax.numpy as jnp
from jax import lax
from jax.experimental import pallas as pl
from jax.experimental.pallas import tpu as pltpu
from jax.experimental.pallas import tpu_sc as plsc

N = 10000
MC = 20000
ME = 320000
NF = 128
ES = 16
C_CORNERS = 3

NCORES = 2
NSUB = 16
NW = NCORES * NSUB

CELLS_PAD = 20480
CPW = CELLS_PAD // NW
CSUB = 128
NCSUB = CPW // CSUB

ME_PAD = 327680
EPW = ME_PAD // NW
ESUB = 128
GRP = 1024
NGRP = EPW // GRP
SPW = EPW // ESUB

PAYW = 24
ACC_ROWS = 10240
RPT = ACC_ROWS // NSUB
DUMMY = N


def _mesh():
    return plsc.VectorSubcoreMesh(
        core_axis_name="c", subcore_axis_name="s",
        num_cores=NCORES, num_subcores=NSUB)


_SC_PARAMS = pltpu.CompilerParams(use_tc_tiling_on_sc=False)


def _sc_cell_mean(V2, cellsc):
    @functools.partial(
        pl.kernel,
        out_type=jax.ShapeDtypeStruct((CELLS_PAD, NF), jnp.float32),
        mesh=_mesh(),
        scratch_types=[
            pltpu.VMEM((3, 128), jnp.int32),
            pltpu.VMEM((CSUB, NF), jnp.float32),
            pltpu.VMEM((CSUB, NF), jnp.float32),
            pltpu.VMEM((CSUB, NF), jnp.float32),
            pltpu.VMEM_SHARED((N, NF), jnp.float32),
            pltpu.SemaphoreType.DMA,
        ],
        compiler_params=_SC_PARAMS,
    )
    def k(v_hbm, cells_hbm, out_hbm, idx_v, r0_v, r1_v, r2_v, v_sp, sem):
        cid = lax.axis_index("c")
        sid = lax.axis_index("s")
        wid = sid * NCORES + cid
        pltpu.sync_copy(v_hbm.at[pl.ds(sid * (N // NSUB), N // NSUB)],
                        v_sp.at[pl.ds(sid * (N // NSUB), N // NSUB)])
        plsc.subcore_barrier()

        def sub(j, carry):
            pltpu.sync_copy(cells_hbm.at[wid, j], idx_v)
            d0 = pltpu.async_copy(v_sp.at[idx_v.at[0]], r0_v, sem)
            d1 = pltpu.async_copy(v_sp.at[idx_v.at[1]], r1_v, sem)
            d2 = pltpu.async_copy(v_sp.at[idx_v.at[2]], r2_v, sem)
            d0.wait()
            d1.wait()
            d2.wait()

            def cell(ci, c2):
                for k8 in range(NF // 16):
                    sl = pl.ds(k8 * 16, 16)
                    r0_v[ci, sl] = (r0_v[ci, sl] + r1_v[ci, sl]
                                    + r2_v[ci, sl]) * (1.0 / 3.0)
                return c2

            lax.fori_loop(0, CSUB, cell, 0, unroll=2)
            pltpu.sync_copy(r0_v,
                            out_hbm.at[pl.ds(wid * CPW + j * CSUB, CSUB)])
            return carry

        lax.fori_loop(0, NCSUB, sub, 0)

    return k(V2, cellsc)


def _sc_edge(pltab, prtab, idxl, idxr):
    RPG = GRP // 8
    RPW = EPW // 8

    @functools.partial(
        pl.kernel,
        out_type=jax.ShapeDtypeStruct((ME_PAD // 8, 128), jnp.float32),
        mesh=_mesh(),
        scratch_types=[
            pltpu.VMEM((GRP // ESUB, 128), jnp.int32),
            pltpu.VMEM((GRP // ESUB, 128), jnp.int32),
            pltpu.VMEM((RPG, 128), jnp.float32),
            pltpu.VMEM((2, ESUB, ES), jnp.float32),
            pltpu.VMEM((2, ESUB, ES), jnp.float32),
            pltpu.VMEM_SHARED((MC, ES), jnp.float32),
            pltpu.VMEM_SHARED((MC, ES), jnp.float32),
            pltpu.SemaphoreType.DMA,
            pltpu.SemaphoreType.DMA,
        ],
        compiler_params=_SC_PARAMS,
    )
    def k(pltab_hbm, prtab_hbm, il_hbm, ir_hbm, x_hbm,
          il_v, ir_v, xout_v, gl_v, gr_v, pl_sp, pr_sp, sem, sem2):
        cid = lax.axis_index("c")
        sid = lax.axis_index("s")
        wid = sid * NCORES + cid

        pltpu.sync_copy(pltab_hbm.at[pl.ds(sid * (MC // NSUB), MC // NSUB)],
                        pl_sp.at[pl.ds(sid * (MC // NSUB), MC // NSUB)])
        pltpu.sync_copy(prtab_hbm.at[pl.ds(sid * (MC // NSUB), MC // NSUB)],
                        pr_sp.at[pl.ds(sid * (MC // NSUB), MC // NSUB)])
        plsc.subcore_barrier()

        NSC = GRP // ESUB

        def grp(g, carry):
            pltpu.sync_copy(il_hbm.at[wid, pl.ds(g * 8, 8)], il_v)
            pltpu.sync_copy(ir_hbm.at[wid, pl.ds(g * 8, 8)], ir_v)
            gd = [None, None]
            gd[0] = (pltpu.async_copy(pl_sp.at[il_v.at[0]], gl_v.at[0], sem),
                     pltpu.async_copy(pr_sp.at[ir_v.at[0]], gr_v.at[0], sem2))
            for j in range(NSC):
                b = j % 2
                dl, dr = gd[b]
                dl.wait()
                dr.wait()
                if j + 1 < NSC:
                    nb = (j + 1) % 2
                    gd[nb] = (
                        pltpu.async_copy(pl_sp.at[il_v.at[j + 1]],
                                         gl_v.at[nb], sem),
                        pltpu.async_copy(pr_sp.at[ir_v.at[j + 1]],
                                         gr_v.at[nb], sem2))

                def edge(r2, c2):
                    for kk in range(8):
                        es = r2 * 8 + kk
                        sl = pl.ds(kk * 16, 16)
                        xout_v[j * 16 + r2, sl] = gl_v[b, es, :] + gr_v[b, es, :]
                    return c2

                lax.fori_loop(0, ESUB // 8, edge, 0, unroll=2)
            pltpu.sync_copy(xout_v,
                            x_hbm.at[pl.ds(wid * RPW + g * RPG, RPG)])
            return carry

        lax.fori_loop(0, NGRP, grp, 0)

    return k(pltab, prtab, idxl, idxr)


def _sc_scatter(h_pk, idx0, idx1):
    RPG = GRP // 8
    RPW = EPW // 8
    out_types = [
        jax.ShapeDtypeStruct((NCORES, ACC_ROWS, PAYW), jnp.float32),
        jax.ShapeDtypeStruct((NCORES, ACC_ROWS, PAYW), jnp.float32),
    ]

    @functools.partial(
        pl.kernel,
        out_type=out_types,
        mesh=_mesh(),
        scratch_types=[
            pltpu.VMEM((GRP // ESUB, 128), jnp.int32),
            pltpu.VMEM((GRP // ESUB, 128), jnp.int32),
            pltpu.VMEM((2, RPG, 128), jnp.float32),
            pltpu.VMEM((2, ESUB, PAYW), jnp.float32),
            pltpu.VMEM((RPT // 5, PAYW), jnp.float32),
            pltpu.VMEM_SHARED((ACC_ROWS, PAYW), jnp.float32),
            pltpu.VMEM_SHARED((ACC_ROWS, PAYW), jnp.float32),
            pltpu.SemaphoreType.DMA,
            pltpu.SemaphoreType.DMA,
            pltpu.SemaphoreType.DMA,
        ],
        compiler_params=_SC_PARAMS,
    )
    def k(h_hbm, i0_hbm, i1_hbm, a0_hbm, a1_hbm,
          i0_v, i1_v, h_v, pay_v, zb_v, acc0, acc1, sem, sem3, sem4):
        cid = lax.axis_index("c")
        sid = lax.axis_index("s")
        wid = sid * NCORES + cid
        ZCH = RPT // 5

        z16 = jnp.zeros((16,), jnp.float32)

        def zrow(i, c):
            zb_v[i, pl.ds(0, 16)] = z16
            zb_v[i, pl.ds(PAYW - 16, 16)] = z16
            return c

        lax.fori_loop(0, ZCH, zrow, 0, unroll=4)

        def zch(i, c):
            pltpu.sync_copy(zb_v, acc0.at[pl.ds(sid * RPT + i * ZCH, ZCH)])
            pltpu.sync_copy(zb_v, acc1.at[pl.ds(sid * RPT + i * ZCH, ZCH)])
            return c

        lax.fori_loop(0, 5, zch, 0)

        o16 = jnp.ones((16,), jnp.float32)

        def prow(i, c):
            pay_v[0, i, pl.ds(PAYW - 16, 16)] = o16
            pay_v[1, i, pl.ds(PAYW - 16, 16)] = o16
            return c

        lax.fori_loop(0, ESUB, prow, 0, unroll=4)
        plsc.subcore_barrier()

        NSC = GRP // ESUB

        def grp(g, carry):
            pltpu.sync_copy(h_hbm.at[pl.ds(wid * RPW + g * RPG, RPG)],
                            h_v.at[0])
            pltpu.sync_copy(i0_hbm.at[wid, pl.ds(g * 8, 8)], i0_v)
            pltpu.sync_copy(i1_hbm.at[wid, pl.ds(g * 8, 8)], i1_v)
            sd = [None, None, None, None]
            for j in range(NSC):
                b = j % 2
                if j >= 2:
                    sd[2 * b].wait()
                    sd[2 * b + 1].wait()

                def edge(r2, c2):
                    for kk in range(8):
                        es = r2 * 8 + kk
                        sl = pl.ds(kk * 16, 16)
                        pay_v[b, es, pl.ds(0, 16)] = h_v[0, j * 16 + r2, sl]
                    return c2

                lax.fori_loop(0, ESUB // 8, edge, 0, unroll=2)
                sd[2 * b] = pltpu.async_copy(pay_v.at[b],
                                             acc0.at[i0_v.at[j]], sem3,
                                             add=True)
                sd[2 * b + 1] = pltpu.async_copy(pay_v.at[b],
                                                 acc1.at[i1_v.at[j]], sem4,
                                                 add=True)
            for j in (NSC - 2, NSC - 1):
                sd[2 * (j % 2)].wait()
                sd[2 * (j % 2) + 1].wait()
            return carry

        lax.fori_loop(0, NGRP, grp, 0)
        plsc.subcore_barrier()

        def dch(i, c):
            pltpu.sync_copy(acc0.at[pl.ds(sid * RPT + i * ZCH, ZCH)], zb_v)
            pltpu.sync_copy(zb_v, a0_hbm.at[cid,
                                            pl.ds(sid * RPT + i * ZCH, ZCH)])
            pltpu.sync_copy(acc1.at[pl.ds(sid * RPT + i * ZCH, ZCH)], zb_v)
            pltpu.sync_copy(zb_v, a1_hbm.at[cid,
                                            pl.ds(sid * RPT + i * ZCH, ZCH)])
            return c

        lax.fori_loop(0, 5, dch, 0)

    return k(h_pk, idx0, idx1)


def _silu(x):
    return x / (1.0 + jnp.exp(-x))


def _dot(a, b):
    return jnp.dot(a, b, preferred_element_type=jnp.float32)


def _tc_edge_fused(s_pk, Epk, w1_blk, b1_blk, w2_blk, b2_blk):
    TILE = 2048
    ROWS = ME_PAD // 8

    def body(s_ref, e_ref, w1_ref, b1_ref, w2_ref, b2_ref, h_ref, o_ref):
        x = s_ref[...] + _dot(e_ref[...], w1_ref[...]) + b1_ref[...]
        h = _silu(x)
        h_ref[...] = h
        o_ref[...] = _dot(h, w2_ref[...]) + b2_ref[...]

    return pl.pallas_call(
        body,
        grid=(ROWS // TILE,),
        in_specs=[
            pl.BlockSpec((TILE, 128), lambda i: (i, 0)),
            pl.BlockSpec((TILE, 128), lambda i: (i, 0)),
            pl.BlockSpec((128, 128), lambda i: (0, 0)),
            pl.BlockSpec((1, 128), lambda i: (0, 0)),
            pl.BlockSpec((128, 128), lambda i: (0, 0)),
            pl.BlockSpec((1, 128), lambda i: (0, 0)),
        ],
        out_specs=[
            pl.BlockSpec((TILE, 128), lambda i: (i, 0)),
            pl.BlockSpec((TILE, 128), lambda i: (i, 0)),
        ],
        out_shape=[
            jax.ShapeDtypeStruct((ROWS, 128), jnp.float32),
            jax.ShapeDtypeStruct((ROWS, 128), jnp.float32),
        ],
    )(s_pk, Epk, w1_blk, b1_blk, w2_blk, b2_blk)


def _tc_cells(cfin, ce_W1, ce_b1, ce_W2, ce_b2, w1l, w1r):
    TILE = 1024

    def body(x_ref, w1_ref, b1_ref, w2_ref, b2_ref, wl_ref, wr_ref,
             pl_ref, pr_ref):
        h = _silu(_dot(x_ref[...], w1_ref[...]) + b1_ref[...])
        cf = _dot(h, w2_ref[...]) + b2_ref[...]
        pl_ref[...] = _dot(cf, wl_ref[...])
        pr_ref[...] = _dot(cf, wr_ref[...])

    return pl.pallas_call(
        body,
        grid=(CELLS_PAD // TILE,),
        in_specs=[
            pl.BlockSpec((TILE, NF), lambda i: (i, 0)),
            pl.BlockSpec((NF, NF), lambda i: (0, 0)),
            pl.BlockSpec((1, NF), lambda i: (0, 0)),
            pl.BlockSpec((NF, NF), lambda i: (0, 0)),
            pl.BlockSpec((1, NF), lambda i: (0, 0)),
            pl.BlockSpec((NF, ES), lambda i: (0, 0)),
            pl.BlockSpec((NF, ES), lambda i: (0, 0)),
        ],
        out_specs=[
            pl.BlockSpec((TILE, ES), lambda i: (i, 0)),
            pl.BlockSpec((TILE, ES), lambda i: (i, 0)),
        ],
        out_shape=[
            jax.ShapeDtypeStruct((CELLS_PAD, ES), jnp.float32),
            jax.ShapeDtypeStruct((CELLS_PAD, ES), jnp.float32),
        ],
    )(cfin, ce_W1, ce_b1, ce_W2, ce_b2, w1l, w1r)


def _tc_nodes(V2, a0d, a1d, p0, q0, p1, q1, w1v, b1, w2, b2):
    TILE = 1000

    def body(v_ref, a0_ref, a1_ref, p0_ref, q0_ref, p1_ref, q1_ref,
             w1_ref, b1_ref, w2_ref, b2_ref, o_ref):
        s0 = a0_ref[0] + a0_ref[1]
        s1 = a1_ref[0] + a1_ref[1]
        c0 = s0[:, 16:17]
        c1 = s1[:, 16:17]
        hm0 = s0[:, 0:16] / jnp.maximum(c0, 1.0)
        hm1 = s1[:, 0:16] / jnp.maximum(c1, 1.0)
        t0 = jnp.where(c0 > 0, _dot(hm0, p0_ref[...]) + q0_ref[...], 0.0)
        t1 = jnp.where(c1 > 0, _dot(hm1, p1_ref[...]) + q1_ref[...], 0.0)
        pre = _dot(v_ref[...], w1_ref[...]) + t0 + t1 + b1_ref[...]
        o_ref[...] = _dot(_silu(pre), w2_ref[...]) + b2_ref[...]

    return pl.pallas_call(
        body,
        grid=(N // TILE,),
        in_specs=[
            pl.BlockSpec((TILE, NF), lambda i: (i, 0)),
            pl.BlockSpec((NCORES, TILE, PAYW), lambda i: (0, i, 0)),
            pl.BlockSpec((NCORES, TILE, PAYW), lambda i: (0, i, 0)),
            pl.BlockSpec((ES, NF), lambda i: (0, 0)),
            pl.BlockSpec((1, NF), lambda i: (0, 0)),
            pl.BlockSpec((ES, NF), lambda i: (0, 0)),
            pl.BlockSpec((1, NF), lambda i: (0, 0)),
            pl.BlockSpec((NF, NF), lambda i: (0, 0)),
            pl.BlockSpec((1, NF), lambda i: (0, 0)),
            pl.BlockSpec((NF, NF), lambda i: (0, 0)),
            pl.BlockSpec((1, NF), lambda i: (0, 0)),
        ],
        out_specs=pl.BlockSpec((TILE, NF), lambda i: (i, 0)),
        out_shape=jax.ShapeDtypeStruct((N, NF), jnp.float32),
    )(V2, a0d, a1d, p0, q0, p1, q1, w1v, b1, w2, b2)


def kernel(V, E, edges, cells, edge_to_cells,
           ce_W1, ce_b1, ce_W2, ce_b2,
           eu_W1, eu_b1, eu_W2, eu_b2,
           nu_W1, nu_b1, nu_W2, nu_b2):
    i32 = jnp.int32
    V2 = V.reshape(N, NF)
    E2 = E.reshape(ME, ES)

    cells2 = jnp.pad(cells.reshape(MC, C_CORNERS).astype(i32),
                     ((0, CELLS_PAD - MC), (0, 0)))
    cellsc = cells2.reshape(NW, NCSUB, CSUB, C_CORNERS).transpose(0, 1, 3, 2)

    lidx = edge_to_cells[0, :, 0].astype(i32)
    ridx = edge_to_cells[0, :, 1].astype(i32)
    lidx2 = jnp.where(lidx >= 0, lidx, ridx)
    ridx2 = jnp.where(ridx >= 0, ridx, lidx)
    pad_e = ME_PAD - ME
    idxl = jnp.pad(lidx2, (0, pad_e)).reshape(NW, SPW, 128)
    idxr = jnp.pad(ridx2, (0, pad_e)).reshape(NW, SPW, 128)
    idx0 = jnp.pad(edges[0, :, 0].astype(i32), (0, pad_e),
                   constant_values=DUMMY).reshape(NW, SPW, 128)
    idx1 = jnp.pad(edges[0, :, 1].astype(i32), (0, pad_e),
                   constant_values=DUMMY).reshape(NW, SPW, 128)
    Epk = jnp.pad(E2, ((0, pad_e), (0, 0))).reshape(ME_PAD // 8, 128)

    w1e = eu_W1[0:ES]
    w1l = eu_W1[ES:ES + NF]
    w1r = eu_W1[ES + NF:]
    eye8 = jnp.eye(8, dtype=jnp.float32)
    w1e_blk = jnp.kron(eye8, w1e)
    b1_blk = jnp.tile(eu_b1, 8).reshape(1, 128)
    w2_blk = jnp.kron(eye8, eu_W2)
    b2_blk = jnp.tile(eu_b2, 8).reshape(1, 128)
    we0 = nu_W1[NF:NF + ES // 2]
    we1 = nu_W1[NF + ES // 2:]
    p0 = eu_W2[:, 0:ES // 2] @ we0
    q0 = (eu_b2[0:ES // 2] @ we0).reshape(1, NF)
    p1 = eu_W2[:, ES // 2:] @ we1
    q1 = (eu_b2[ES // 2:] @ we1).reshape(1, NF)
    w1v = nu_W1[0:NF]

    cfin = _sc_cell_mean(V2, cellsc)

    pltab, prtab = _tc_cells(cfin, ce_W1, ce_b1.reshape(1, NF),
                             ce_W2, ce_b2.reshape(1, NF), w1l, w1r)

    s_pk = _sc_edge(pltab, prtab, idxl, idxr)

    h_pk, edge_emb_pk = _tc_edge_fused(s_pk, Epk, w1e_blk, b1_blk,
                                       w2_blk, b2_blk)
    edge_emb = edge_emb_pk.reshape(ME_PAD, ES)[:ME]

    a0d, a1d = _sc_scatter(h_pk, idx0, idx1)

    node_emb = _tc_nodes(V2, a0d, a1d, p0, q0, p1, q1, w1v,
                         nu_b1.reshape(1, NF), nu_W2, nu_b2.reshape(1, NF))

    return (node_emb.reshape(1, N, NF), edge_emb.reshape(1, ME, ES))


# speedup vs baseline: 18.9527x; 1.2419x over previous
import functools

import jax
import jax.numpy as jnp
from jax import lax
from jax.experimental import pallas as pl
from jax.experimental.pallas import tpu as pltpu
from jax.experimental.pallas import tpu_sc as plsc

N = 10000
MC = 20000
ME = 320000
NF = 128
ES = 16
C_CORNERS = 3

NCORES = 2
NSUB = 16
NW = NCORES * NSUB

CELLS_PAD = 20480
CPW = CELLS_PAD // NW
CSUB = 128
NCSUB = CPW // CSUB

ME_PAD = 327680
EPW = ME_PAD // NW
ESUB = 128
GRP = 1024
NGRP = EPW // GRP
SPW = EPW // ESUB

PAYW = 24
ACC_ROWS = 10240
RPT = ACC_ROWS // NSUB
DUMMY = N


def _mesh():
    return plsc.VectorSubcoreMesh(
        core_axis_name="c", subcore_axis_name="s",
        num_cores=NCORES, num_subcores=NSUB)


_SC_PARAMS = pltpu.CompilerParams(use_tc_tiling_on_sc=False)


def _sc_cell_mean(V2, cellsc):
    @functools.partial(
        pl.kernel,
        out_type=jax.ShapeDtypeStruct((CELLS_PAD, NF), jnp.float32),
        mesh=_mesh(),
        scratch_types=[
            pltpu.VMEM((3, 128), jnp.int32),
            pltpu.VMEM((CSUB, NF), jnp.float32),
            pltpu.VMEM((CSUB, NF), jnp.float32),
            pltpu.VMEM((CSUB, NF), jnp.float32),
            pltpu.VMEM_SHARED((N, NF), jnp.float32),
            pltpu.SemaphoreType.DMA,
        ],
        compiler_params=_SC_PARAMS,
    )
    def k(v_hbm, cells_hbm, out_hbm, idx_v, r0_v, r1_v, r2_v, v_sp, sem):
        cid = lax.axis_index("c")
        sid = lax.axis_index("s")
        wid = sid * NCORES + cid
        pltpu.sync_copy(v_hbm.at[pl.ds(sid * (N // NSUB), N // NSUB)],
                        v_sp.at[pl.ds(sid * (N // NSUB), N // NSUB)])
        plsc.subcore_barrier()

        def sub(j, carry):
            pltpu.sync_copy(cells_hbm.at[wid, j], idx_v)
            d0 = pltpu.async_copy(v_sp.at[idx_v.at[0]], r0_v, sem)
            d1 = pltpu.async_copy(v_sp.at[idx_v.at[1]], r1_v, sem)
            d2 = pltpu.async_copy(v_sp.at[idx_v.at[2]], r2_v, sem)
            d0.wait()
            d1.wait()
            d2.wait()

            def cell(ci, c2):
                for k8 in range(NF // 16):
                    sl = pl.ds(k8 * 16, 16)
                    r0_v[ci, sl] = (r0_v[ci, sl] + r1_v[ci, sl]
                                    + r2_v[ci, sl]) * (1.0 / 3.0)
                return c2

            lax.fori_loop(0, CSUB, cell, 0, unroll=2)
            pltpu.sync_copy(r0_v,
                            out_hbm.at[pl.ds(wid * CPW + j * CSUB, CSUB)])
            return carry

        lax.fori_loop(0, NCSUB, sub, 0)

    return k(V2, cellsc)


def _sc_edge(pltab, prtab, idxl, idxr):
    RPG = GRP // 8
    RPW = EPW // 8

    @functools.partial(
        pl.kernel,
        out_type=jax.ShapeDtypeStruct((ME_PAD // 8, 128), jnp.float32),
        mesh=_mesh(),
        scratch_types=[
            pltpu.VMEM((GRP // ESUB, 128), jnp.int32),
            pltpu.VMEM((GRP // ESUB, 128), jnp.int32),
            pltpu.VMEM((RPG, 128), jnp.float32),
            pltpu.VMEM((2, ESUB, ES), jnp.float32),
            pltpu.VMEM((2, ESUB, ES), jnp.float32),
            pltpu.VMEM_SHARED((MC, ES), jnp.float32),
            pltpu.VMEM_SHARED((MC, ES), jnp.float32),
            pltpu.SemaphoreType.DMA,
            pltpu.SemaphoreType.DMA,
        ],
        compiler_params=_SC_PARAMS,
    )
    def k(pltab_hbm, prtab_hbm, il_hbm, ir_hbm, x_hbm,
          il_v, ir_v, xout_v, gl_v, gr_v, pl_sp, pr_sp, sem, sem2):
        cid = lax.axis_index("c")
        sid = lax.axis_index("s")
        wid = sid * NCORES + cid

        pltpu.sync_copy(pltab_hbm.at[pl.ds(sid * (MC // NSUB), MC // NSUB)],
                        pl_sp.at[pl.ds(sid * (MC // NSUB), MC // NSUB)])
        pltpu.sync_copy(prtab_hbm.at[pl.ds(sid * (MC // NSUB), MC // NSUB)],
                        pr_sp.at[pl.ds(sid * (MC // NSUB), MC // NSUB)])
        plsc.subcore_barrier()

        NSC = GRP // ESUB

        def grp(g, carry):
            pltpu.sync_copy(il_hbm.at[wid, pl.ds(g * 8, 8)], il_v)
            pltpu.sync_copy(ir_hbm.at[wid, pl.ds(g * 8, 8)], ir_v)
            gd = [None, None]
            gd[0] = (pltpu.async_copy(pl_sp.at[il_v.at[0]], gl_v.at[0], sem),
                     pltpu.async_copy(pr_sp.at[ir_v.at[0]], gr_v.at[0], sem2))
            for j in range(NSC):
                b = j % 2
                dl, dr = gd[b]
                dl.wait()
                dr.wait()
                if j + 1 < NSC:
                    nb = (j + 1) % 2
                    gd[nb] = (
                        pltpu.async_copy(pl_sp.at[il_v.at[j + 1]],
                                         gl_v.at[nb], sem),
                        pltpu.async_copy(pr_sp.at[ir_v.at[j + 1]],
                                         gr_v.at[nb], sem2))

                def edge(r2, c2):
                    for kk in range(8):
                        es = r2 * 8 + kk
                        sl = pl.ds(kk * 16, 16)
                        xout_v[j * 16 + r2, sl] = gl_v[b, es, :] + gr_v[b, es, :]
                    return c2

                lax.fori_loop(0, ESUB // 8, edge, 0, unroll=2)
            pltpu.sync_copy(xout_v,
                            x_hbm.at[pl.ds(wid * RPW + g * RPG, RPG)])
            return carry

        lax.fori_loop(0, NGRP, grp, 0)

    return k(pltab, prtab, idxl, idxr)


def _sc_scatter(h_pk, idx0, idx1):
    RPG = GRP // 8
    RPW = EPW // 8
    out_types = [
        jax.ShapeDtypeStruct((NCORES, ACC_ROWS, PAYW), jnp.float32),
        jax.ShapeDtypeStruct((NCORES, ACC_ROWS, PAYW), jnp.float32),
    ]

    @functools.partial(
        pl.kernel,
        out_type=out_types,
        mesh=_mesh(),
        scratch_types=[
            pltpu.VMEM((GRP // ESUB, 128), jnp.int32),
            pltpu.VMEM((GRP // ESUB, 128), jnp.int32),
            pltpu.VMEM((2, RPG, 128), jnp.float32),
            pltpu.VMEM((2, ESUB, PAYW), jnp.float32),
            pltpu.VMEM((RPT // 5, PAYW), jnp.float32),
            pltpu.VMEM_SHARED((ACC_ROWS, PAYW), jnp.float32),
            pltpu.VMEM_SHARED((ACC_ROWS, PAYW), jnp.float32),
            pltpu.SemaphoreType.DMA,
            pltpu.SemaphoreType.DMA,
            pltpu.SemaphoreType.DMA,
        ],
        compiler_params=_SC_PARAMS,
    )
    def k(h_hbm, i0_hbm, i1_hbm, a0_hbm, a1_hbm,
          i0_v, i1_v, h_v, pay_v, zb_v, acc0, acc1, sem, sem3, sem4):
        cid = lax.axis_index("c")
        sid = lax.axis_index("s")
        wid = sid * NCORES + cid
        ZCH = RPT // 5

        z16 = jnp.zeros((16,), jnp.float32)

        def zrow(i, c):
            zb_v[i, pl.ds(0, 16)] = z16
            zb_v[i, pl.ds(PAYW - 16, 16)] = z16
            return c

        lax.fori_loop(0, ZCH, zrow, 0, unroll=4)

        def zch(i, c):
            pltpu.sync_copy(zb_v, acc0.at[pl.ds(sid * RPT + i * ZCH, ZCH)])
            pltpu.sync_copy(zb_v, acc1.at[pl.ds(sid * RPT + i * ZCH, ZCH)])
            return c

        lax.fori_loop(0, 5, zch, 0)

        o16 = jnp.ones((16,), jnp.float32)

        def prow(i, c):
            pay_v[0, i, pl.ds(PAYW - 16, 16)] = o16
            pay_v[1, i, pl.ds(PAYW - 16, 16)] = o16
            return c

        lax.fori_loop(0, ESUB, prow, 0, unroll=4)
        plsc.subcore_barrier()

        NSC = GRP // ESUB

        def grp(g, carry):
            pltpu.sync_copy(h_hbm.at[pl.ds(wid * RPW + g * RPG, RPG)],
                            h_v.at[0])
            pltpu.sync_copy(i0_hbm.at[wid, pl.ds(g * 8, 8)], i0_v)
            pltpu.sync_copy(i1_hbm.at[wid, pl.ds(g * 8, 8)], i1_v)
            sd = [None, None, None, None]
            for j in range(NSC):
                b = j % 2
                if j >= 2:
                    sd[2 * b].wait()
                    sd[2 * b + 1].wait()

                def edge(r2, c2):
                    for kk in range(8):
                        es = r2 * 8 + kk
                        sl = pl.ds(kk * 16, 16)
                        pay_v[b, es, pl.ds(0, 16)] = h_v[0, j * 16 + r2, sl]
                    return c2

                lax.fori_loop(0, ESUB // 8, edge, 0, unroll=2)
                sd[2 * b] = pltpu.async_copy(pay_v.at[b],
                                             acc0.at[i0_v.at[j]], sem3,
                                             add=True)
                sd[2 * b + 1] = pltpu.async_copy(pay_v.at[b],
                                                 acc1.at[i1_v.at[j]], sem4,
                                                 add=True)
            for j in (NSC - 2, NSC - 1):
                sd[2 * (j % 2)].wait()
                sd[2 * (j % 2) + 1].wait()
            return carry

        lax.fori_loop(0, NGRP, grp, 0)
        plsc.subcore_barrier()

        def dch(i, c):
            pltpu.sync_copy(acc0.at[pl.ds(sid * RPT + i * ZCH, ZCH)], zb_v)
            pltpu.sync_copy(zb_v, a0_hbm.at[cid,
                                            pl.ds(sid * RPT + i * ZCH, ZCH)])
            pltpu.sync_copy(acc1.at[pl.ds(sid * RPT + i * ZCH, ZCH)], zb_v)
            pltpu.sync_copy(zb_v, a1_hbm.at[cid,
                                            pl.ds(sid * RPT + i * ZCH, ZCH)])
            return c

        lax.fori_loop(0, 5, dch, 0)

    return k(h_pk, idx0, idx1)


def _silu(x):
    return x / (1.0 + jnp.exp(-x))


def _dot(a, b):
    return jnp.dot(a, b, preferred_element_type=jnp.float32)


def _tc_edge_fused(s_pk, Epk, w1_blk, b1_blk, w2_blk, b2_blk):
    TILE = 2000
    ROWS = ME // 8

    def body(s_ref, e_ref, w1_ref, b1_ref, w2_ref, b2_ref, h_ref, o_ref):
        x = s_ref[...] + _dot(e_ref[...], w1_ref[...]) + b1_ref[...]
        h = _silu(x)
        h_ref[...] = h
        o_ref[...] = _dot(h, w2_ref[...]) + b2_ref[...]

    return pl.pallas_call(
        body,
        grid=(ROWS // TILE,),
        in_specs=[
            pl.BlockSpec((TILE, 128), lambda i: (i, 0)),
            pl.BlockSpec((TILE, 128), lambda i: (i, 0)),
            pl.BlockSpec((128, 128), lambda i: (0, 0)),
            pl.BlockSpec((1, 128), lambda i: (0, 0)),
            pl.BlockSpec((128, 128), lambda i: (0, 0)),
            pl.BlockSpec((1, 128), lambda i: (0, 0)),
        ],
        out_specs=[
            pl.BlockSpec((TILE, 128), lambda i: (i, 0)),
            pl.BlockSpec((TILE, 128), lambda i: (i, 0)),
        ],
        out_shape=[
            jax.ShapeDtypeStruct((ME_PAD // 8, 128), jnp.float32),
            jax.ShapeDtypeStruct((ME_PAD // 8, 128), jnp.float32),
        ],
    )(s_pk, Epk, w1_blk, b1_blk, w2_blk, b2_blk)


def _tc_cells(cfin, ce_W1, ce_b1, ce_W2, ce_b2, w1l, w1r):
    TILE = 1024

    def body(x_ref, w1_ref, b1_ref, w2_ref, b2_ref, wl_ref, wr_ref,
             pl_ref, pr_ref):
        h = _silu(_dot(x_ref[...], w1_ref[...]) + b1_ref[...])
        cf = _dot(h, w2_ref[...]) + b2_ref[...]
        pl_ref[...] = _dot(cf, wl_ref[...])
        pr_ref[...] = _dot(cf, wr_ref[...])

    return pl.pallas_call(
        body,
        grid=(CELLS_PAD // TILE,),
        in_specs=[
            pl.BlockSpec((TILE, NF), lambda i: (i, 0)),
            pl.BlockSpec((NF, NF), lambda i: (0, 0)),
            pl.BlockSpec((1, NF), lambda i: (0, 0)),
            pl.BlockSpec((NF, NF), lambda i: (0, 0)),
            pl.BlockSpec((1, NF), lambda i: (0, 0)),
            pl.BlockSpec((NF, ES), lambda i: (0, 0)),
            pl.BlockSpec((NF, ES), lambda i: (0, 0)),
        ],
        out_specs=[
            pl.BlockSpec((TILE, ES), lambda i: (i, 0)),
            pl.BlockSpec((TILE, ES), lambda i: (i, 0)),
        ],
        out_shape=[
            jax.ShapeDtypeStruct((CELLS_PAD, ES), jnp.float32),
            jax.ShapeDtypeStruct((CELLS_PAD, ES), jnp.float32),
        ],
    )(cfin, ce_W1, ce_b1, ce_W2, ce_b2, w1l, w1r)


def _tc_nodes(V2, a0d, a1d, p0, q0, p1, q1, w1v, b1, w2, b2):
    TILE = 1000

    def body(v_ref, a0_ref, a1_ref, p0_ref, q0_ref, p1_ref, q1_ref,
             w1_ref, b1_ref, w2_ref, b2_ref, o_ref):
        s0 = a0_ref[0] + a0_ref[1]
        s1 = a1_ref[0] + a1_ref[1]
        c0 = s0[:, 16:17]
        c1 = s1[:, 16:17]
        hm0 = s0[:, 0:16] / jnp.maximum(c0, 1.0)
        hm1 = s1[:, 0:16] / jnp.maximum(c1, 1.0)
        t0 = jnp.where(c0 > 0, _dot(hm0, p0_ref[...]) + q0_ref[...], 0.0)
        t1 = jnp.where(c1 > 0, _dot(hm1, p1_ref[...]) + q1_ref[...], 0.0)
        pre = _dot(v_ref[...], w1_ref[...]) + t0 + t1 + b1_ref[...]
        o_ref[...] = _dot(_silu(pre), w2_ref[...]) + b2_ref[...]

    return pl.pallas_call(
        body,
        grid=(N // TILE,),
        in_specs=[
            pl.BlockSpec((TILE, NF), lambda i: (i, 0)),
            pl.BlockSpec((NCORES, TILE, PAYW), lambda i: (0, i, 0)),
            pl.BlockSpec((NCORES, TILE, PAYW), lambda i: (0, i, 0)),
            pl.BlockSpec((ES, NF), lambda i: (0, 0)),
            pl.BlockSpec((1, NF), lambda i: (0, 0)),
            pl.BlockSpec((ES, NF), lambda i: (0, 0)),
            pl.BlockSpec((1, NF), lambda i: (0, 0)),
            pl.BlockSpec((NF, NF), lambda i: (0, 0)),
            pl.BlockSpec((1, NF), lambda i: (0, 0)),
            pl.BlockSpec((NF, NF), lambda i: (0, 0)),
            pl.BlockSpec((1, NF), lambda i: (0, 0)),
        ],
        out_specs=pl.BlockSpec((TILE, NF), lambda i: (i, 0)),
        out_shape=jax.ShapeDtypeStruct((N, NF), jnp.float32),
    )(V2, a0d, a1d, p0, q0, p1, q1, w1v, b1, w2, b2)


def kernel(V, E, edges, cells, edge_to_cells,
           ce_W1, ce_b1, ce_W2, ce_b2,
           eu_W1, eu_b1, eu_W2, eu_b2,
           nu_W1, nu_b1, nu_W2, nu_b2):
    i32 = jnp.int32
    V2 = V.reshape(N, NF)
    E2 = E.reshape(ME, ES)

    cells2 = jnp.pad(cells.reshape(MC, C_CORNERS).astype(i32),
                     ((0, CELLS_PAD - MC), (0, 0)))
    cellsc = cells2.reshape(NW, NCSUB, CSUB, C_CORNERS).transpose(0, 1, 3, 2)

    lidx = edge_to_cells[0, :, 0].astype(i32)
    ridx = edge_to_cells[0, :, 1].astype(i32)
    lidx2 = jnp.where(lidx >= 0, lidx, ridx)
    ridx2 = jnp.where(ridx >= 0, ridx, lidx)
    pad_e = ME_PAD - ME
    idxl = jnp.pad(lidx2, (0, pad_e)).reshape(NW, SPW, 128)
    idxr = jnp.pad(ridx2, (0, pad_e)).reshape(NW, SPW, 128)
    idx0 = jnp.pad(edges[0, :, 0].astype(i32), (0, pad_e),
                   constant_values=DUMMY).reshape(NW, SPW, 128)
    idx1 = jnp.pad(edges[0, :, 1].astype(i32), (0, pad_e),
                   constant_values=DUMMY).reshape(NW, SPW, 128)
    Epk = E2.reshape(ME // 8, 128)

    w1e = eu_W1[0:ES]
    w1l = eu_W1[ES:ES + NF]
    w1r = eu_W1[ES + NF:]
    eye8 = jnp.eye(8, dtype=jnp.float32)
    w1e_blk = jnp.kron(eye8, w1e)
    b1_blk = jnp.tile(eu_b1, 8).reshape(1, 128)
    w2_blk = jnp.kron(eye8, eu_W2)
    b2_blk = jnp.tile(eu_b2, 8).reshape(1, 128)
    we0 = nu_W1[NF:NF + ES // 2]
    we1 = nu_W1[NF + ES // 2:]
    p0 = eu_W2[:, 0:ES // 2] @ we0
    q0 = (eu_b2[0:ES // 2] @ we0).reshape(1, NF)
    p1 = eu_W2[:, ES // 2:] @ we1
    q1 = (eu_b2[ES // 2:] @ we1).reshape(1, NF)
    w1v = nu_W1[0:NF]

    cfin = _sc_cell_mean(V2, cellsc)

    pltab, prtab = _tc_cells(cfin, ce_W1, ce_b1.reshape(1, NF),
                             ce_W2, ce_b2.reshape(1, NF), w1l, w1r)

    s_pk = _sc_edge(pltab, prtab, idxl, idxr)

    h_pk, edge_emb_pk = _tc_edge_fused(s_pk, Epk, w1e_blk, b1_blk,
                                       w2_blk, b2_blk)
    edge_emb = edge_emb_pk.reshape(ME_PAD, ES)[:ME]

    a0d, a1d = _sc_scatter(h_pk, idx0, idx1)

    node_emb = _tc_nodes(V2, a0d, a1d, p0, q0, p1, q1, w1v,
                         nu_b1.reshape(1, NF), nu_W2, nu_b2.reshape(1, NF))

    return (node_emb.reshape(1, N, NF), edge_emb.reshape(1, ME, ES))
